# Initial kernel scaffold; baseline (speedup 1.0000x reference)
import jax
import jax.numpy as jnp
from jax.experimental import pallas as pl


def kernel(x, edge_index, edge_attr, rbf, sbf, edge_index_0, W_rbf0, W_rbf1, W_sbf0, W_sbf1, W_ek, W_ev, W_k, b_k, W_q, b_q, W_v, b_v, W_skip, b_skip):
    raise NotImplementedError("write your pallas kernel here")


# speedup vs baseline: 3.5730x; 3.5730x over previous
import math

import jax
import jax.numpy as jnp
from jax import lax
from jax.experimental import pallas as pl
from jax.experimental.pallas import tpu as pltpu
from jax.experimental.pallas import tpu_sc as plsc

N = 10000
E = 320000
D = 128
C = 128
ED = 16
SBF6 = 42

NC = 2
NS = 16
NW = NC * NS
CHUNK = 128
CPW = 79
E2 = NW * CPW * CHUNK
NPAD = 10240
RPS = NPAD // NS

_SQRT_C_INV = 1.0 / math.sqrt(C)


def _node_kernel(x_ref, rbf_ref, wr0_ref, wr1_ref, wk_ref, bk_ref, wq_ref,
                 bq_ref, wv_ref, bv_ref, ws_ref, bs_ref,
                 k_ref, q_ref, v_ref, skip_ref):
    x = x_ref[...]
    rbf_f = jnp.dot(jnp.dot(rbf_ref[...], wr0_ref[...],
                            preferred_element_type=jnp.float32),
                    wr1_ref[...], preferred_element_type=jnp.float32)
    x_src = rbf_f * x
    k_ref[...] = jnp.dot(x_src, wk_ref[...],
                         preferred_element_type=jnp.float32) + bk_ref[...]
    q_ref[...] = jnp.dot(x, wq_ref[...],
                         preferred_element_type=jnp.float32) + bq_ref[...]
    v_ref[...] = jnp.dot(x_src, wv_ref[...],
                         preferred_element_type=jnp.float32) + bv_ref[...]
    skip_ref[...] = jnp.dot(x, ws_ref[...],
                            preferred_element_type=jnp.float32) + bs_ref[...]


def _node_pass(x_p, rbf_p, W_rbf0, W_rbf1, W_k, b_k, W_q, b_q, W_v, b_v,
               W_skip, b_skip):
    B = 512
    g = NPAD // B
    row = lambda i: (i, 0)
    full = lambda i: (0, 0)
    nd = jax.ShapeDtypeStruct((NPAD, D), jnp.float32)
    return pl.pallas_call(
        _node_kernel,
        grid=(g,),
        in_specs=[
            pl.BlockSpec((B, D), row),
            pl.BlockSpec((B, SBF6), row),
            pl.BlockSpec((SBF6, D), full),
            pl.BlockSpec((D, D), full),
            pl.BlockSpec((D, D), full),
            pl.BlockSpec((1, D), full),
            pl.BlockSpec((D, D), full),
            pl.BlockSpec((1, D), full),
            pl.BlockSpec((D, D), full),
            pl.BlockSpec((1, D), full),
            pl.BlockSpec((D, D), full),
            pl.BlockSpec((1, D), full),
        ],
        out_specs=[pl.BlockSpec((B, D), row)] * 4,
        out_shape=[nd, nd, nd, nd],
    )(x_p, rbf_p, W_rbf0, W_rbf1, W_k, b_k.reshape(1, D), W_q,
      b_q.reshape(1, D), W_v, b_v.reshape(1, D), W_skip,
      b_skip.reshape(1, D))


def _gather_body(ktab, qtab, vtab, src_h, dst_h,
                 ksrc_o, qdst_o, vsrc_o,
                 idx_s, idx_d, buf_k, buf_q, buf_v,
                 sem0, sem1, sem2):
    wid = lax.axis_index("s") * NC + lax.axis_index("c")

    def chunk(j, _):
        base = (wid * CPW + j) * CHUNK
        pltpu.sync_copy(src_h.at[pl.ds(base, CHUNK)], idx_s)
        pltpu.sync_copy(dst_h.at[pl.ds(base, CHUNK)], idx_d)
        c0 = pltpu.async_copy(ktab.at[idx_s], buf_k, sem0)
        c1 = pltpu.async_copy(qtab.at[idx_d], buf_q, sem1)
        c2 = pltpu.async_copy(vtab.at[idx_s], buf_v, sem2)
        c0.wait()
        c1.wait()
        c2.wait()
        pltpu.sync_copy(buf_k, ksrc_o.at[pl.ds(base, CHUNK)])
        pltpu.sync_copy(buf_q, qdst_o.at[pl.ds(base, CHUNK)])
        pltpu.sync_copy(buf_v, vsrc_o.at[pl.ds(base, CHUNK)])
        return 0

    lax.fori_loop(0, CPW, chunk, 0)


def _gather_pass(ktab, qtab, vtab, src_p, dst_p):
    mesh = plsc.VectorSubcoreMesh(core_axis_name="c", subcore_axis_name="s")
    ed = jax.ShapeDtypeStruct((E2, D), jnp.float32)
    fn = pl.kernel(
        _gather_body,
        out_type=(ed, ed, ed),
        mesh=mesh,
        scratch_types=[
            pltpu.VMEM((CHUNK,), jnp.int32),
            pltpu.VMEM((CHUNK,), jnp.int32),
            pltpu.VMEM((CHUNK, D), jnp.float32),
            pltpu.VMEM((CHUNK, D), jnp.float32),
            pltpu.VMEM((CHUNK, D), jnp.float32),
            pltpu.SemaphoreType.DMA,
            pltpu.SemaphoreType.DMA,
            pltpu.SemaphoreType.DMA,
        ],
    )
    return fn(ktab, qtab, vtab, src_p, dst_p)


def _edge_kernel(ks_ref, qd_ref, vs_ref, ea_ref, sbf_ref,
                 wekt_ref, wev_ref, ws0_ref, ws1_ref, msg_ref, eal_ref):
    ks = ks_ref[...]
    qd = qd_ref[...]
    ea = ea_ref[...]
    qw = jnp.dot(qd, wekt_ref[...], preferred_element_type=jnp.float32)
    alpha = (jnp.sum(ks * qd, axis=1, keepdims=True)
             + jnp.sum(qw * ea, axis=1, keepdims=True)) * _SQRT_C_INV
    ealpha = jnp.exp(alpha)
    ev = jnp.dot(ea, wev_ref[...], preferred_element_type=jnp.float32)
    sb = jnp.dot(jnp.dot(sbf_ref[...], ws0_ref[...],
                         preferred_element_type=jnp.float32),
                 ws1_ref[...], preferred_element_type=jnp.float32)
    msg_ref[...] = (vs_ref[...] + ev) * sb * ealpha
    eal_ref[...] = ealpha


def _edge_pass(ksrc, qdst, vsrc, ea_p, sbf_p, W_ekT, W_ev, W_sbf0, W_sbf1):
    B = 512
    g = E2 // B
    row = lambda i: (i, 0)
    full = lambda i: (0, 0)
    return pl.pallas_call(
        _edge_kernel,
        grid=(g,),
        in_specs=[
            pl.BlockSpec((B, D), row),
            pl.BlockSpec((B, D), row),
            pl.BlockSpec((B, D), row),
            pl.BlockSpec((B, ED), row),
            pl.BlockSpec((B, ED), row),
            pl.BlockSpec((D, ED), full),
            pl.BlockSpec((ED, D), full),
            pl.BlockSpec((ED, D), full),
            pl.BlockSpec((D, D), full),
        ],
        out_specs=[
            pl.BlockSpec((B, D), row),
            pl.BlockSpec((B, 1), row),
        ],
        out_shape=[
            jax.ShapeDtypeStruct((E2, D), jnp.float32),
            jax.ShapeDtypeStruct((E2, 1), jnp.float32),
        ],
    )(ksrc, qdst, vsrc, ea_p, sbf_p, W_ekT, W_ev, W_sbf0, W_sbf1)


def _scatter_body(msg_h, eal_h, dst_h, zeros_h, acc_o, den_o,
                  idx_d, mbuf, ebuf, tbuf, dacc, denom, acc, den_sh):
    cid = lax.axis_index("c")
    sid = lax.axis_index("s")
    wid = sid * NC + cid

    if True:
        def initz(s, _):
            r0 = sid * RPS + s * CHUNK
            pltpu.sync_copy(zeros_h.at[pl.ds(r0, CHUNK)],
                            acc.at[pl.ds(r0, CHUNK)])
            return 0

        def initd(i, _):
            denom[pl.ds(i * 16, 16)] = jnp.zeros((16,), jnp.float32)
            return 0

        lax.fori_loop(0, RPS // CHUNK, initz, 0)
        lax.fori_loop(0, NPAD // 16, initd, 0)
        plsc.subcore_barrier()

        def chunk(j, _):
            base = (wid * CPW + j) * CHUNK
            pltpu.sync_copy(dst_h.at[pl.ds(base, CHUNK)], idx_d)
            pltpu.sync_copy(msg_h.at[pl.ds(base, CHUNK)], mbuf)
            pltpu.sync_copy(eal_h.at[pl.ds(base, CHUNK)], ebuf)
            pltpu.sync_copy(mbuf, acc.at[idx_d], add=True)
            for k in range(CHUNK // 16):
                i16 = idx_d[pl.ds(k * 16, 16)]
                e16 = ebuf[pl.ds(k * 16, 16)]
                plsc.addupdate_scatter(denom, [i16], e16)
            return 0

        lax.fori_loop(0, CPW, chunk, 0)

        pltpu.sync_copy(denom, den_sh.at[sid])
        plsc.subcore_barrier()

        def zslice(i, _):
            dacc[pl.ds(i * 16, 16)] = jnp.zeros((16,), jnp.float32)
            return 0

        lax.fori_loop(0, RPS // 16, zslice, 0)

        def redt(t, _):
            pltpu.sync_copy(den_sh.at[t, pl.ds(sid * RPS, RPS)], tbuf)

            def addv(i, _):
                dacc[pl.ds(i * 16, 16)] = (dacc[pl.ds(i * 16, 16)]
                                           + tbuf[pl.ds(i * 16, 16)])
                return 0

            lax.fori_loop(0, RPS // 16, addv, 0)
            return 0

        lax.fori_loop(0, NS, redt, 0)

        r0 = sid * RPS
        pltpu.sync_copy(acc.at[pl.ds(r0, RPS)],
                        acc_o.at[cid, pl.ds(r0, RPS)])
        pltpu.sync_copy(dacc, den_o.at[cid, pl.ds(r0, RPS)])


def _scatter_pass(msg, eal, dst_p, zeros_nd):
    mesh = plsc.VectorSubcoreMesh(core_axis_name="c", subcore_axis_name="s")
    fn = pl.kernel(
        _scatter_body,
        out_type=(
            jax.ShapeDtypeStruct((NC, NPAD, D), jnp.float32),
            jax.ShapeDtypeStruct((NC, NPAD), jnp.float32),
        ),
        mesh=mesh,
        scratch_types=[
            pltpu.VMEM((CHUNK,), jnp.int32),
            pltpu.VMEM((CHUNK, D), jnp.float32),
            pltpu.VMEM((CHUNK,), jnp.float32),
            pltpu.VMEM((RPS,), jnp.float32),
            pltpu.VMEM((RPS,), jnp.float32),
            pltpu.VMEM((NPAD,), jnp.float32),
            pltpu.VMEM_SHARED((NPAD, D), jnp.float32),
            pltpu.VMEM_SHARED((NS, NPAD), jnp.float32),
        ],
        compiler_params=pltpu.CompilerParams(needs_layout_passes=False),
    )
    return fn(msg, eal, dst_p, zeros_nd)


def _final_kernel(a0_ref, a1_ref, d0_ref, d1_ref, skip_ref, out_ref):
    den = d0_ref[...] + d1_ref[...] + 1e-16
    out_ref[...] = (a0_ref[...] + a1_ref[...]) / den + skip_ref[...]


def _final_pass(a0, a1, d0, d1, skip):
    B = 512
    g = pl.cdiv(N, B)
    row = lambda i: (i, 0)
    return pl.pallas_call(
        _final_kernel,
        grid=(g,),
        in_specs=[
            pl.BlockSpec((B, D), row),
            pl.BlockSpec((B, D), row),
            pl.BlockSpec((B, 1), row),
            pl.BlockSpec((B, 1), row),
            pl.BlockSpec((B, D), row),
        ],
        out_specs=pl.BlockSpec((B, D), row),
        out_shape=jax.ShapeDtypeStruct((N, D), jnp.float32),
    )(a0, a1, d0, d1, skip)


def kernel(x, edge_index, edge_attr, rbf, sbf, edge_index_0,
           W_rbf0, W_rbf1, W_sbf0, W_sbf1, W_ek, W_ev,
           W_k, b_k, W_q, b_q, W_v, b_v, W_skip, b_skip):
    pad_e = E2 - E
    src_p = jnp.concatenate(
        [edge_index[0], jnp.zeros((pad_e,), jnp.int32)])
    dst_p = jnp.concatenate(
        [edge_index[1], jnp.full((pad_e,), N, jnp.int32)])
    ea_p = jnp.pad(edge_attr, ((0, pad_e), (0, 0)))
    sbf_p = jnp.pad(sbf.reshape(E, ED), ((0, pad_e), (0, 0)))
    x_p = jnp.pad(x, ((0, NPAD - N), (0, 0)))
    rbf_p = jnp.pad(rbf, ((0, NPAD - N), (0, 0)))

    ktab, qtab, vtab, skip = _node_pass(
        x_p, rbf_p, W_rbf0, W_rbf1, W_k, b_k, W_q, b_q, W_v, b_v,
        W_skip, b_skip)

    ksrc, qdst, vsrc = _gather_pass(ktab, qtab, vtab, src_p, dst_p)

    msg, eal = _edge_pass(ksrc, qdst, vsrc, ea_p, sbf_p,
                          W_ek.T, W_ev, W_sbf0, W_sbf1)

    zeros_nd = jnp.zeros((NPAD, D), jnp.float32)
    accs, dens = _scatter_pass(msg, eal.reshape(E2), dst_p, zeros_nd)

    return _final_pass(accs[0], accs[1],
                       dens[0, :N].reshape(N, 1), dens[1, :N].reshape(N, 1),
                       skip[:N])


# speedup vs baseline: 3.6878x; 1.0321x over previous
import math

import jax
import jax.numpy as jnp
from jax import lax
from jax.experimental import pallas as pl
from jax.experimental.pallas import tpu as pltpu
from jax.experimental.pallas import tpu_sc as plsc

N = 10000
E = 320000
D = 128
C = 128
ED = 16
SBF6 = 42

NC = 2
NS = 16
NW = NC * NS
CHUNK = 128
CPW = 79
E2 = NW * CPW * CHUNK
NPAD = 10240
RPS = NPAD // NS

_SQRT_C_INV = 1.0 / math.sqrt(C)


def _node_kernel(x_ref, rbf_ref, wr0_ref, wr1_ref, wk_ref, bk_ref, wq_ref,
                 bq_ref, wv_ref, bv_ref, ws_ref, bs_ref,
                 k_ref, q_ref, v_ref, skip_ref):
    x = x_ref[...]
    rbf_f = jnp.dot(jnp.dot(rbf_ref[...], wr0_ref[...],
                            preferred_element_type=jnp.float32),
                    wr1_ref[...], preferred_element_type=jnp.float32)
    x_src = rbf_f * x
    k_ref[...] = jnp.dot(x_src, wk_ref[...],
                         preferred_element_type=jnp.float32) + bk_ref[...]
    q_ref[...] = jnp.dot(x, wq_ref[...],
                         preferred_element_type=jnp.float32) + bq_ref[...]
    v_ref[...] = jnp.dot(x_src, wv_ref[...],
                         preferred_element_type=jnp.float32) + bv_ref[...]
    skip_ref[...] = jnp.dot(x, ws_ref[...],
                            preferred_element_type=jnp.float32) + bs_ref[...]


def _node_pass(x_p, rbf_p, W_rbf0, W_rbf1, W_k, b_k, W_q, b_q, W_v, b_v,
               W_skip, b_skip):
    B = 512
    g = NPAD // B
    row = lambda i: (i, 0)
    full = lambda i: (0, 0)
    nd = jax.ShapeDtypeStruct((NPAD, D), jnp.float32)
    return pl.pallas_call(
        _node_kernel,
        grid=(g,),
        in_specs=[
            pl.BlockSpec((B, D), row),
            pl.BlockSpec((B, SBF6), row),
            pl.BlockSpec((SBF6, D), full),
            pl.BlockSpec((D, D), full),
            pl.BlockSpec((D, D), full),
            pl.BlockSpec((1, D), full),
            pl.BlockSpec((D, D), full),
            pl.BlockSpec((1, D), full),
            pl.BlockSpec((D, D), full),
            pl.BlockSpec((1, D), full),
            pl.BlockSpec((D, D), full),
            pl.BlockSpec((1, D), full),
        ],
        out_specs=[pl.BlockSpec((B, D), row)] * 4,
        out_shape=[nd, nd, nd, nd],
    )(x_p, rbf_p, W_rbf0, W_rbf1, W_k, b_k.reshape(1, D), W_q,
      b_q.reshape(1, D), W_v, b_v.reshape(1, D), W_skip,
      b_skip.reshape(1, D))


EPW = CPW * CHUNK


def _gather_body(ktab, qtab, vtab, src_h, dst_h,
                 ksrc_o, qdst_o, vsrc_o,
                 idx_s, idx_d, bk0, bq0, bv0, bk1, bq1, bv1,
                 gs0, gs1, ws0, ws1):
    wid = lax.axis_index("s") * NC + lax.axis_index("c")
    e0 = wid * EPW
    pltpu.sync_copy(src_h.at[pl.ds(e0, EPW)], idx_s)
    pltpu.sync_copy(dst_h.at[pl.ds(e0, EPW)], idx_d)

    bufs = ((bk0, bq0, bv0), (bk1, bq1, bv1))
    gsems = (gs0, gs1)
    wsems = (ws0, ws1)

    def fire_gathers(j, s):
        o = j * CHUNK
        i_s = idx_s.at[pl.ds(o, CHUNK)]
        i_d = idx_d.at[pl.ds(o, CHUNK)]
        pltpu.async_copy(ktab.at[i_s], bufs[s][0], gsems[s])
        pltpu.async_copy(qtab.at[i_d], bufs[s][1], gsems[s])
        pltpu.async_copy(vtab.at[i_s], bufs[s][2], gsems[s])

    def drain_gathers(s):
        for b in bufs[s]:
            pltpu.make_async_copy(ktab.at[idx_s.at[pl.ds(0, CHUNK)]],
                                  b, gsems[s]).wait()

    def fire_writes(j, s):
        base = wid * EPW + j * CHUNK
        pltpu.async_copy(bufs[s][0], ksrc_o.at[pl.ds(base, CHUNK)], wsems[s])
        pltpu.async_copy(bufs[s][1], qdst_o.at[pl.ds(base, CHUNK)], wsems[s])
        pltpu.async_copy(bufs[s][2], vsrc_o.at[pl.ds(base, CHUNK)], wsems[s])

    def drain_writes(s):
        for b in bufs[s]:
            pltpu.make_async_copy(b, ksrc_o.at[pl.ds(0, CHUNK)],
                                  wsems[s]).wait()

    fire_gathers(0, 0)

    def body(j, _):
        s = lax.rem(j, 2)
        ns = 1 - s

        @pl.when(j >= 1)
        def _():
            @pl.when(ns == 0)
            def _():
                drain_writes(0)

            @pl.when(ns == 1)
            def _():
                drain_writes(1)

        @pl.when(j + 1 < CPW)
        def _():
            @pl.when(ns == 0)
            def _():
                fire_gathers(j + 1, 0)

            @pl.when(ns == 1)
            def _():
                fire_gathers(j + 1, 1)

        @pl.when(s == 0)
        def _():
            drain_gathers(0)
            fire_writes(j, 0)

        @pl.when(s == 1)
        def _():
            drain_gathers(1)
            fire_writes(j, 1)

        return 0

    lax.fori_loop(0, CPW, body, 0)
    drain_writes((CPW - 1) % 2)


def _gather_pass(ktab, qtab, vtab, src_p, dst_p):
    mesh = plsc.VectorSubcoreMesh(core_axis_name="c", subcore_axis_name="s")
    ed = jax.ShapeDtypeStruct((E2, D), jnp.float32)
    buf = pltpu.VMEM((CHUNK, D), jnp.float32)
    fn = pl.kernel(
        _gather_body,
        out_type=(ed, ed, ed),
        mesh=mesh,
        scratch_types=[
            pltpu.VMEM((EPW,), jnp.int32),
            pltpu.VMEM((EPW,), jnp.int32),
            buf, buf, buf, buf, buf, buf,
            pltpu.SemaphoreType.DMA,
            pltpu.SemaphoreType.DMA,
            pltpu.SemaphoreType.DMA,
            pltpu.SemaphoreType.DMA,
        ],
    )
    return fn(ktab, qtab, vtab, src_p, dst_p)


def _edge_kernel(ks_ref, qd_ref, vs_ref, ea_ref, sbf_ref,
                 wekt_ref, wev_ref, ws0_ref, ws1_ref, msg_ref, eal_ref):
    ks = ks_ref[...]
    qd = qd_ref[...]
    ea = ea_ref[...]
    qw = jnp.dot(qd, wekt_ref[...], preferred_element_type=jnp.float32)
    alpha = (jnp.sum(ks * qd, axis=1, keepdims=True)
             + jnp.sum(qw * ea, axis=1, keepdims=True)) * _SQRT_C_INV
    ealpha = jnp.exp(alpha)
    ev = jnp.dot(ea, wev_ref[...], preferred_element_type=jnp.float32)
    sb = jnp.dot(jnp.dot(sbf_ref[...], ws0_ref[...],
                         preferred_element_type=jnp.float32),
                 ws1_ref[...], preferred_element_type=jnp.float32)
    msg_ref[...] = (vs_ref[...] + ev) * sb * ealpha
    eal_ref[...] = ealpha


def _edge_pass(ksrc, qdst, vsrc, ea_p, sbf_p, W_ekT, W_ev, W_sbf0, W_sbf1):
    B = 512
    g = E2 // B
    row = lambda i: (i, 0)
    full = lambda i: (0, 0)
    return pl.pallas_call(
        _edge_kernel,
        grid=(g,),
        in_specs=[
            pl.BlockSpec((B, D), row),
            pl.BlockSpec((B, D), row),
            pl.BlockSpec((B, D), row),
            pl.BlockSpec((B, ED), row),
            pl.BlockSpec((B, ED), row),
            pl.BlockSpec((D, ED), full),
            pl.BlockSpec((ED, D), full),
            pl.BlockSpec((ED, D), full),
            pl.BlockSpec((D, D), full),
        ],
        out_specs=[
            pl.BlockSpec((B, D), row),
            pl.BlockSpec((B, 1), row),
        ],
        out_shape=[
            jax.ShapeDtypeStruct((E2, D), jnp.float32),
            jax.ShapeDtypeStruct((E2, 1), jnp.float32),
        ],
    )(ksrc, qdst, vsrc, ea_p, sbf_p, W_ekT, W_ev, W_sbf0, W_sbf1)


def _scatter_body(msg_h, eal_h, dst_h, zeros_h, acc_o, den_o,
                  idx_d, mbuf, ebuf, tbuf, dacc, denom, acc, den_sh):
    cid = lax.axis_index("c")
    sid = lax.axis_index("s")
    wid = sid * NC + cid

    if True:
        def initz(s, _):
            r0 = sid * RPS + s * CHUNK
            pltpu.sync_copy(zeros_h.at[pl.ds(r0, CHUNK)],
                            acc.at[pl.ds(r0, CHUNK)])
            return 0

        def initd(i, _):
            denom[pl.ds(i * 16, 16)] = jnp.zeros((16,), jnp.float32)
            return 0

        lax.fori_loop(0, RPS // CHUNK, initz, 0)
        lax.fori_loop(0, NPAD // 16, initd, 0)
        plsc.subcore_barrier()

        def chunk(j, _):
            base = (wid * CPW + j) * CHUNK
            pltpu.sync_copy(dst_h.at[pl.ds(base, CHUNK)], idx_d)
            pltpu.sync_copy(msg_h.at[pl.ds(base, CHUNK)], mbuf)
            pltpu.sync_copy(eal_h.at[pl.ds(base, CHUNK)], ebuf)
            pltpu.sync_copy(mbuf, acc.at[idx_d], add=True)
            for k in range(CHUNK // 16):
                i16 = idx_d[pl.ds(k * 16, 16)]
                e16 = ebuf[pl.ds(k * 16, 16)]
                plsc.addupdate_scatter(denom, [i16], e16)
            return 0

        lax.fori_loop(0, CPW, chunk, 0)

        pltpu.sync_copy(denom, den_sh.at[sid])
        plsc.subcore_barrier()

        def zslice(i, _):
            dacc[pl.ds(i * 16, 16)] = jnp.zeros((16,), jnp.float32)
            return 0

        lax.fori_loop(0, RPS // 16, zslice, 0)

        def redt(t, _):
            pltpu.sync_copy(den_sh.at[t, pl.ds(sid * RPS, RPS)], tbuf)

            def addv(i, _):
                dacc[pl.ds(i * 16, 16)] = (dacc[pl.ds(i * 16, 16)]
                                           + tbuf[pl.ds(i * 16, 16)])
                return 0

            lax.fori_loop(0, RPS // 16, addv, 0)
            return 0

        lax.fori_loop(0, NS, redt, 0)

        r0 = sid * RPS
        pltpu.sync_copy(acc.at[pl.ds(r0, RPS)],
                        acc_o.at[cid, pl.ds(r0, RPS)])
        pltpu.sync_copy(dacc, den_o.at[cid, pl.ds(r0, RPS)])


def _scatter_pass(msg, eal, dst_p, zeros_nd):
    mesh = plsc.VectorSubcoreMesh(core_axis_name="c", subcore_axis_name="s")
    fn = pl.kernel(
        _scatter_body,
        out_type=(
            jax.ShapeDtypeStruct((NC, NPAD, D), jnp.float32),
            jax.ShapeDtypeStruct((NC, NPAD), jnp.float32),
        ),
        mesh=mesh,
        scratch_types=[
            pltpu.VMEM((CHUNK,), jnp.int32),
            pltpu.VMEM((CHUNK, D), jnp.float32),
            pltpu.VMEM((CHUNK,), jnp.float32),
            pltpu.VMEM((RPS,), jnp.float32),
            pltpu.VMEM((RPS,), jnp.float32),
            pltpu.VMEM((NPAD,), jnp.float32),
            pltpu.VMEM_SHARED((NPAD, D), jnp.float32),
            pltpu.VMEM_SHARED((NS, NPAD), jnp.float32),
        ],
        compiler_params=pltpu.CompilerParams(needs_layout_passes=False),
    )
    return fn(msg, eal, dst_p, zeros_nd)


def _final_kernel(a0_ref, a1_ref, d0_ref, d1_ref, skip_ref, out_ref):
    a = a0_ref[0] + a1_ref[0]
    den = d0_ref[0] + d1_ref[0] + 1e-16
    out_ref[...] = a / den + skip_ref[...]


def _final_pass(accs, dens3, skip):
    B = 512
    g = pl.cdiv(N, B)
    row = lambda i: (i, 0)
    return pl.pallas_call(
        _final_kernel,
        grid=(g,),
        in_specs=[
            pl.BlockSpec((1, B, D), lambda i: (0, i, 0)),
            pl.BlockSpec((1, B, D), lambda i: (1, i, 0)),
            pl.BlockSpec((1, B, 1), lambda i: (0, i, 0)),
            pl.BlockSpec((1, B, 1), lambda i: (1, i, 0)),
            pl.BlockSpec((B, D), row),
        ],
        out_specs=pl.BlockSpec((B, D), row),
        out_shape=jax.ShapeDtypeStruct((N, D), jnp.float32),
    )(accs, accs, dens3, dens3, skip)


def kernel(x, edge_index, edge_attr, rbf, sbf, edge_index_0,
           W_rbf0, W_rbf1, W_sbf0, W_sbf1, W_ek, W_ev,
           W_k, b_k, W_q, b_q, W_v, b_v, W_skip, b_skip):
    pad_e = E2 - E
    src_p = jnp.concatenate(
        [edge_index[0], jnp.zeros((pad_e,), jnp.int32)])
    dst_p = jnp.concatenate(
        [edge_index[1], jnp.full((pad_e,), N, jnp.int32)])
    ea_p = jnp.pad(edge_attr, ((0, pad_e), (0, 0)))
    sbf_p = jnp.pad(sbf.reshape(E, ED), ((0, pad_e), (0, 0)))
    x_p = jnp.pad(x, ((0, NPAD - N), (0, 0)))
    rbf_p = jnp.pad(rbf, ((0, NPAD - N), (0, 0)))

    ktab, qtab, vtab, skip = _node_pass(
        x_p, rbf_p, W_rbf0, W_rbf1, W_k, b_k, W_q, b_q, W_v, b_v,
        W_skip, b_skip)

    ksrc, qdst, vsrc = _gather_pass(ktab, qtab, vtab, src_p, dst_p)

    msg, eal = _edge_pass(ksrc, qdst, vsrc, ea_p, sbf_p,
                          W_ek.T, W_ev, W_sbf0, W_sbf1)

    zeros_nd = jnp.zeros((NPAD, D), jnp.float32)
    accs, dens = _scatter_pass(msg, eal.reshape(E2), dst_p, zeros_nd)

    return _final_pass(accs, dens.reshape(NC, NPAD, 1), skip)


# speedup vs baseline: 4.2777x; 1.1599x over previous
import math

import jax
import jax.numpy as jnp
from jax import lax
from jax.experimental import pallas as pl
from jax.experimental.pallas import tpu as pltpu
from jax.experimental.pallas import tpu_sc as plsc

N = 10000
E = 320000
D = 128
C = 128
ED = 16
SBF6 = 42

NC = 2
NS = 16
NW = NC * NS
CHUNK = 128
CPW = 79
E2 = NW * CPW * CHUNK
NPAD = 10240
RPS = NPAD // NS

_SQRT_C_INV = 1.0 / math.sqrt(C)


def _node_kernel(x_ref, rbf_ref, wr0_ref, wr1_ref, wk_ref, bk_ref, wq_ref,
                 bq_ref, wv_ref, bv_ref, ws_ref, bs_ref,
                 k_ref, q_ref, v_ref, skip_ref):
    x = x_ref[...]
    rbf_f = jnp.dot(jnp.dot(rbf_ref[...], wr0_ref[...],
                            preferred_element_type=jnp.float32),
                    wr1_ref[...], preferred_element_type=jnp.float32)
    x_src = rbf_f * x
    k_ref[...] = jnp.dot(x_src, wk_ref[...],
                         preferred_element_type=jnp.float32) + bk_ref[...]
    q_ref[...] = jnp.dot(x, wq_ref[...],
                         preferred_element_type=jnp.float32) + bq_ref[...]
    v_ref[...] = jnp.dot(x_src, wv_ref[...],
                         preferred_element_type=jnp.float32) + bv_ref[...]
    skip_ref[...] = jnp.dot(x, ws_ref[...],
                            preferred_element_type=jnp.float32) + bs_ref[...]


def _node_pass(x_p, rbf_p, W_rbf0, W_rbf1, W_k, b_k, W_q, b_q, W_v, b_v,
               W_skip, b_skip):
    B = 512
    g = NPAD // B
    row = lambda i: (i, 0)
    full = lambda i: (0, 0)
    nd = jax.ShapeDtypeStruct((NPAD, D), jnp.float32)
    return pl.pallas_call(
        _node_kernel,
        grid=(g,),
        in_specs=[
            pl.BlockSpec((B, D), row),
            pl.BlockSpec((B, SBF6), row),
            pl.BlockSpec((SBF6, D), full),
            pl.BlockSpec((D, D), full),
            pl.BlockSpec((D, D), full),
            pl.BlockSpec((1, D), full),
            pl.BlockSpec((D, D), full),
            pl.BlockSpec((1, D), full),
            pl.BlockSpec((D, D), full),
            pl.BlockSpec((1, D), full),
            pl.BlockSpec((D, D), full),
            pl.BlockSpec((1, D), full),
        ],
        out_specs=[pl.BlockSpec((B, D), row)] * 4,
        out_shape=[nd, nd, nd, nd],
    )(x_p, rbf_p, W_rbf0, W_rbf1, W_k, b_k.reshape(1, D), W_q,
      b_q.reshape(1, D), W_v, b_v.reshape(1, D), W_skip,
      b_skip.reshape(1, D))


def _gather_body(cpw, e_base, ktab, qtab, vtab, src_h, dst_h,
                 ksrc_o, qdst_o, vsrc_o,
                 idx_s, idx_d, bk0, bq0, bv0, bk1, bq1, bv1,
                 gs0, gs1, ws0, ws1):
    epw = cpw * CHUNK
    wid = lax.axis_index("s") * NC + lax.axis_index("c")
    e0 = e_base + wid * epw
    pltpu.sync_copy(src_h.at[pl.ds(e0, epw)], idx_s)
    pltpu.sync_copy(dst_h.at[pl.ds(e0, epw)], idx_d)

    bufs = ((bk0, bq0, bv0), (bk1, bq1, bv1))
    gsems = (gs0, gs1)
    wsems = (ws0, ws1)

    def fire_gathers(j, s):
        o = j * CHUNK
        i_s = idx_s.at[pl.ds(o, CHUNK)]
        i_d = idx_d.at[pl.ds(o, CHUNK)]
        pltpu.async_copy(ktab.at[i_s], bufs[s][0], gsems[s])
        pltpu.async_copy(qtab.at[i_d], bufs[s][1], gsems[s])
        pltpu.async_copy(vtab.at[i_s], bufs[s][2], gsems[s])

    def drain_gathers(s):
        for b in bufs[s]:
            pltpu.make_async_copy(ktab.at[idx_s.at[pl.ds(0, CHUNK)]],
                                  b, gsems[s]).wait()

    def fire_writes(j, s):
        base = wid * epw + j * CHUNK
        pltpu.async_copy(bufs[s][0], ksrc_o.at[pl.ds(base, CHUNK)], wsems[s])
        pltpu.async_copy(bufs[s][1], qdst_o.at[pl.ds(base, CHUNK)], wsems[s])
        pltpu.async_copy(bufs[s][2], vsrc_o.at[pl.ds(base, CHUNK)], wsems[s])

    def drain_writes(s):
        for b in bufs[s]:
            pltpu.make_async_copy(b, ksrc_o.at[pl.ds(0, CHUNK)],
                                  wsems[s]).wait()

    fire_gathers(0, 0)

    def body(j, _):
        s = lax.rem(j, 2)
        ns = 1 - s

        @pl.when(j >= 1)
        def _():
            @pl.when(ns == 0)
            def _():
                drain_writes(0)

            @pl.when(ns == 1)
            def _():
                drain_writes(1)

        @pl.when(j + 1 < cpw)
        def _():
            @pl.when(ns == 0)
            def _():
                fire_gathers(j + 1, 0)

            @pl.when(ns == 1)
            def _():
                fire_gathers(j + 1, 1)

        @pl.when(s == 0)
        def _():
            drain_gathers(0)
            fire_writes(j, 0)

        @pl.when(s == 1)
        def _():
            drain_gathers(1)
            fire_writes(j, 1)

        return 0

    lax.fori_loop(0, cpw, body, 0)
    drain_writes((cpw - 1) % 2)


def _gather_pass(ktab, qtab, vtab, src_p, dst_p, e_base, cpw, ne):
    import functools
    mesh = plsc.VectorSubcoreMesh(core_axis_name="c", subcore_axis_name="s")
    ed = jax.ShapeDtypeStruct((ne, D), jnp.float32)
    buf = pltpu.VMEM((CHUNK, D), jnp.float32)
    fn = pl.kernel(
        functools.partial(_gather_body, cpw, e_base),
        out_type=(ed, ed, ed),
        mesh=mesh,
        scratch_types=[
            pltpu.VMEM((cpw * CHUNK,), jnp.int32),
            pltpu.VMEM((cpw * CHUNK,), jnp.int32),
            buf, buf, buf, buf, buf, buf,
            pltpu.SemaphoreType.DMA,
            pltpu.SemaphoreType.DMA,
            pltpu.SemaphoreType.DMA,
            pltpu.SemaphoreType.DMA,
        ],
    )
    return fn(ktab, qtab, vtab, src_p, dst_p)


def _edge_kernel(ks_ref, qd_ref, vs_ref, ea_ref, sbf_ref,
                 wekt_ref, wev_ref, ws0_ref, ws1_ref, msg_ref, eal_ref):
    ks = ks_ref[...]
    qd = qd_ref[...]
    ea = ea_ref[...]
    qw = jnp.dot(qd, wekt_ref[...], preferred_element_type=jnp.float32)
    alpha = (jnp.sum(ks * qd, axis=1, keepdims=True)
             + jnp.sum(qw * ea, axis=1, keepdims=True)) * _SQRT_C_INV
    ealpha = jnp.exp(alpha)
    ev = jnp.dot(ea, wev_ref[...], preferred_element_type=jnp.float32)
    sb = jnp.dot(jnp.dot(sbf_ref[...], ws0_ref[...],
                         preferred_element_type=jnp.float32),
                 ws1_ref[...], preferred_element_type=jnp.float32)
    msg_ref[...] = (vs_ref[...] + ev) * sb * ealpha
    eal_ref[...] = ealpha


def _edge_pass(ksrc, qdst, vsrc, ea_p, sbf_p, W_ekT, W_ev, W_sbf0, W_sbf1,
               e_base):
    B = 512
    ne = ksrc.shape[0]
    g = ne // B
    ob = e_base // B
    row = lambda i: (i, 0)
    rowo = lambda i: (i + ob, 0)
    full = lambda i: (0, 0)
    return pl.pallas_call(
        _edge_kernel,
        grid=(g,),
        in_specs=[
            pl.BlockSpec((B, D), row),
            pl.BlockSpec((B, D), row),
            pl.BlockSpec((B, D), row),
            pl.BlockSpec((B, ED), rowo),
            pl.BlockSpec((B, ED), rowo),
            pl.BlockSpec((D, ED), full),
            pl.BlockSpec((ED, D), full),
            pl.BlockSpec((ED, D), full),
            pl.BlockSpec((D, D), full),
        ],
        out_specs=[
            pl.BlockSpec((B, D), row),
            pl.BlockSpec((B, 1), row),
        ],
        out_shape=[
            jax.ShapeDtypeStruct((ne, D), jnp.float32),
            jax.ShapeDtypeStruct((ne, 1), jnp.float32),
        ],
    )(ksrc, qdst, vsrc, ea_p, sbf_p, W_ekT, W_ev, W_sbf0, W_sbf1)


def _scatter_body(cpw, e_base, msg_h, eal_h, dst_h, zeros_h, acc_o, den_o,
                  idx_d, mbuf, ebuf, tbuf, dacc, denom, acc, den_sh):
    cid = lax.axis_index("c")
    sid = lax.axis_index("s")
    wid = sid * NC + cid

    if True:
        def initz(s, _):
            r0 = sid * RPS + s * CHUNK
            pltpu.sync_copy(zeros_h.at[pl.ds(r0, CHUNK)],
                            acc.at[pl.ds(r0, CHUNK)])
            return 0

        def initd(i, _):
            denom[pl.ds(i * 16, 16)] = jnp.zeros((16,), jnp.float32)
            return 0

        lax.fori_loop(0, RPS // CHUNK, initz, 0)
        lax.fori_loop(0, NPAD // 16, initd, 0)
        plsc.subcore_barrier()

        def chunk(j, _):
            lbase = (wid * cpw + j) * CHUNK
            pltpu.sync_copy(dst_h.at[pl.ds(e_base + lbase, CHUNK)], idx_d)
            pltpu.sync_copy(msg_h.at[pl.ds(lbase, CHUNK)], mbuf)
            pltpu.sync_copy(eal_h.at[pl.ds(lbase, CHUNK)], ebuf)
            pltpu.sync_copy(mbuf, acc.at[idx_d], add=True)
            for k in range(CHUNK // 16):
                i16 = idx_d[pl.ds(k * 16, 16)]
                e16 = ebuf[pl.ds(k * 16, 16)]
                plsc.addupdate_scatter(denom, [i16], e16)
            return 0

        lax.fori_loop(0, cpw, chunk, 0)

        pltpu.sync_copy(denom, den_sh.at[sid])
        plsc.subcore_barrier()

        def zslice(i, _):
            dacc[pl.ds(i * 16, 16)] = jnp.zeros((16,), jnp.float32)
            return 0

        lax.fori_loop(0, RPS // 16, zslice, 0)

        def redt(t, _):
            pltpu.sync_copy(den_sh.at[t, pl.ds(sid * RPS, RPS)], tbuf)

            def addv(i, _):
                dacc[pl.ds(i * 16, 16)] = (dacc[pl.ds(i * 16, 16)]
                                           + tbuf[pl.ds(i * 16, 16)])
                return 0

            lax.fori_loop(0, RPS // 16, addv, 0)
            return 0

        lax.fori_loop(0, NS, redt, 0)

        r0 = sid * RPS
        pltpu.sync_copy(acc.at[pl.ds(r0, RPS)],
                        acc_o.at[cid, pl.ds(r0, RPS)])
        pltpu.sync_copy(dacc, den_o.at[cid, pl.ds(r0, RPS)])


def _scatter_pass(msg, eal, dst_p, zeros_nd, e_base, cpw):
    import functools
    mesh = plsc.VectorSubcoreMesh(core_axis_name="c", subcore_axis_name="s")
    fn = pl.kernel(
        functools.partial(_scatter_body, cpw, e_base),
        out_type=(
            jax.ShapeDtypeStruct((NC, NPAD, D), jnp.float32),
            jax.ShapeDtypeStruct((NC, NPAD), jnp.float32),
        ),
        mesh=mesh,
        scratch_types=[
            pltpu.VMEM((CHUNK,), jnp.int32),
            pltpu.VMEM((CHUNK, D), jnp.float32),
            pltpu.VMEM((CHUNK,), jnp.float32),
            pltpu.VMEM((RPS,), jnp.float32),
            pltpu.VMEM((RPS,), jnp.float32),
            pltpu.VMEM((NPAD,), jnp.float32),
            pltpu.VMEM_SHARED((NPAD, D), jnp.float32),
            pltpu.VMEM_SHARED((NS, NPAD), jnp.float32),
        ],
        compiler_params=pltpu.CompilerParams(needs_layout_passes=False),
    )
    return fn(msg, eal, dst_p, zeros_nd)


def _final_kernel(a0_ref, a1_ref, a2_ref, a3_ref,
                  d0_ref, d1_ref, d2_ref, d3_ref, skip_ref, out_ref):
    a = (a0_ref[0] + a1_ref[0]) + (a2_ref[0] + a3_ref[0])
    den = (d0_ref[0] + d1_ref[0]) + (d2_ref[0] + d3_ref[0]) + 1e-16
    out_ref[...] = a / den + skip_ref[...]


def _final_pass(accsA, accsB, dens3A, dens3B, skip):
    B = 512
    g = pl.cdiv(N, B)
    row = lambda i: (i, 0)
    c0 = lambda i: (0, i, 0)
    c1 = lambda i: (1, i, 0)
    return pl.pallas_call(
        _final_kernel,
        grid=(g,),
        in_specs=[
            pl.BlockSpec((1, B, D), c0),
            pl.BlockSpec((1, B, D), c1),
            pl.BlockSpec((1, B, D), c0),
            pl.BlockSpec((1, B, D), c1),
            pl.BlockSpec((1, B, 1), c0),
            pl.BlockSpec((1, B, 1), c1),
            pl.BlockSpec((1, B, 1), c0),
            pl.BlockSpec((1, B, 1), c1),
            pl.BlockSpec((B, D), row),
        ],
        out_specs=pl.BlockSpec((B, D), row),
        out_shape=jax.ShapeDtypeStruct((N, D), jnp.float32),
    )(accsA, accsA, accsB, accsB, dens3A, dens3A, dens3B, dens3B, skip)


def kernel(x, edge_index, edge_attr, rbf, sbf, edge_index_0,
           W_rbf0, W_rbf1, W_sbf0, W_sbf1, W_ek, W_ev,
           W_k, b_k, W_q, b_q, W_v, b_v, W_skip, b_skip):
    pad_e = E2 - E
    src_p = jnp.concatenate(
        [edge_index[0], jnp.zeros((pad_e,), jnp.int32)])
    dst_p = jnp.concatenate(
        [edge_index[1], jnp.full((pad_e,), N, jnp.int32)])
    ea_p = jnp.pad(edge_attr, ((0, pad_e), (0, 0)))
    sbf_p = jnp.pad(sbf.reshape(E, ED), ((0, pad_e), (0, 0)))
    x_p = jnp.pad(x, ((0, NPAD - N), (0, 0)))
    rbf_p = jnp.pad(rbf, ((0, NPAD - N), (0, 0)))

    ktab, qtab, vtab, skip = _node_pass(
        x_p, rbf_p, W_rbf0, W_rbf1, W_k, b_k, W_q, b_q, W_v, b_v,
        W_skip, b_skip)

    CPW_A, CPW_B = 40, 39
    EA_N = NW * CPW_A * CHUNK
    EB_N = NW * CPW_B * CHUNK
    W_ekT = W_ek.T
    zeros_nd = jnp.zeros((NPAD, D), jnp.float32)

    gA = _gather_pass(ktab, qtab, vtab, src_p, dst_p, 0, CPW_A, EA_N)
    gB = _gather_pass(ktab, qtab, vtab, src_p, dst_p, EA_N, CPW_B, EB_N)

    msgA, ealA = _edge_pass(*gA, ea_p, sbf_p, W_ekT, W_ev, W_sbf0, W_sbf1, 0)
    msgB, ealB = _edge_pass(*gB, ea_p, sbf_p, W_ekT, W_ev, W_sbf0, W_sbf1,
                            EA_N)

    accsA, densA = _scatter_pass(msgA, ealA.reshape(EA_N), dst_p, zeros_nd,
                                 0, CPW_A)
    accsB, densB = _scatter_pass(msgB, ealB.reshape(EB_N), dst_p, zeros_nd,
                                 EA_N, CPW_B)

    return _final_pass(accsA, accsB,
                       densA.reshape(NC, NPAD, 1), densB.reshape(NC, NPAD, 1),
                       skip)


# speedup vs baseline: 4.7382x; 1.1076x over previous
import math

import jax
import jax.numpy as jnp
from jax import lax
from jax.experimental import pallas as pl
from jax.experimental.pallas import tpu as pltpu
from jax.experimental.pallas import tpu_sc as plsc

N = 10000
E = 320000
D = 128
C = 128
ED = 16
SBF6 = 42

NC = 2
NS = 16
NW = NC * NS
CHUNK = 128
CPW = 79
E2 = NW * CPW * CHUNK
NPAD = 10240
RPS = NPAD // NS

_SQRT_C_INV = 1.0 / math.sqrt(C)


def _node_kernel(x_ref, rbf_ref, wr0_ref, wr1_ref, wk_ref, bk_ref, wq_ref,
                 bq_ref, wv_ref, bv_ref, ws_ref, bs_ref,
                 k_ref, q_ref, v_ref, skip_ref):
    x = x_ref[...]
    rbf_f = jnp.dot(jnp.dot(rbf_ref[...], wr0_ref[...],
                            preferred_element_type=jnp.float32),
                    wr1_ref[...], preferred_element_type=jnp.float32)
    x_src = rbf_f * x
    k_ref[...] = jnp.dot(x_src, wk_ref[...],
                         preferred_element_type=jnp.float32) + bk_ref[...]
    q_ref[...] = jnp.dot(x, wq_ref[...],
                         preferred_element_type=jnp.float32) + bq_ref[...]
    v_ref[...] = jnp.dot(x_src, wv_ref[...],
                         preferred_element_type=jnp.float32) + bv_ref[...]
    skip_ref[...] = jnp.dot(x, ws_ref[...],
                            preferred_element_type=jnp.float32) + bs_ref[...]


def _node_pass(x_p, rbf_p, W_rbf0, W_rbf1, W_k, b_k, W_q, b_q, W_v, b_v,
               W_skip, b_skip):
    B = 512
    g = NPAD // B
    row = lambda i: (i, 0)
    full = lambda i: (0, 0)
    nd = jax.ShapeDtypeStruct((NPAD, D), jnp.float32)
    return pl.pallas_call(
        _node_kernel,
        grid=(g,),
        in_specs=[
            pl.BlockSpec((B, D), row),
            pl.BlockSpec((B, SBF6), row),
            pl.BlockSpec((SBF6, D), full),
            pl.BlockSpec((D, D), full),
            pl.BlockSpec((D, D), full),
            pl.BlockSpec((1, D), full),
            pl.BlockSpec((D, D), full),
            pl.BlockSpec((1, D), full),
            pl.BlockSpec((D, D), full),
            pl.BlockSpec((1, D), full),
            pl.BlockSpec((D, D), full),
            pl.BlockSpec((1, D), full),
        ],
        out_specs=[pl.BlockSpec((B, D), row)] * 4,
        out_shape=[nd, nd, nd, nd],
    )(x_p, rbf_p, W_rbf0, W_rbf1, W_k, b_k.reshape(1, D), W_q,
      b_q.reshape(1, D), W_v, b_v.reshape(1, D), W_skip,
      b_skip.reshape(1, D))


def _gather_body(cpw, e_base, ktab, qtab, vtab, src_h, dst_h,
                 ksrc_o, qdst_o, vsrc_o,
                 idx_s, idx_d, bk0, bq0, bv0, bk1, bq1, bv1,
                 gs0, gs1, ws0, ws1):
    epw = cpw * CHUNK
    wid = lax.axis_index("s") * NC + lax.axis_index("c")
    e0 = e_base + wid * epw
    pltpu.sync_copy(src_h.at[pl.ds(e0, epw)], idx_s)
    pltpu.sync_copy(dst_h.at[pl.ds(e0, epw)], idx_d)

    bufs = ((bk0, bq0, bv0), (bk1, bq1, bv1))
    gsems = (gs0, gs1)
    wsems = (ws0, ws1)

    def fire_gathers(j, s):
        o = j * CHUNK
        i_s = idx_s.at[pl.ds(o, CHUNK)]
        i_d = idx_d.at[pl.ds(o, CHUNK)]
        pltpu.async_copy(ktab.at[i_s], bufs[s][0], gsems[s])
        pltpu.async_copy(qtab.at[i_d], bufs[s][1], gsems[s])
        pltpu.async_copy(vtab.at[i_s], bufs[s][2], gsems[s])

    def drain_gathers(s):
        for b in bufs[s]:
            pltpu.make_async_copy(ktab.at[idx_s.at[pl.ds(0, CHUNK)]],
                                  b, gsems[s]).wait()

    def fire_writes(j, s):
        base = wid * epw + j * CHUNK
        pltpu.async_copy(bufs[s][0], ksrc_o.at[pl.ds(base, CHUNK)], wsems[s])
        pltpu.async_copy(bufs[s][1], qdst_o.at[pl.ds(base, CHUNK)], wsems[s])
        pltpu.async_copy(bufs[s][2], vsrc_o.at[pl.ds(base, CHUNK)], wsems[s])

    def drain_writes(s):
        for b in bufs[s]:
            pltpu.make_async_copy(b, ksrc_o.at[pl.ds(0, CHUNK)],
                                  wsems[s]).wait()

    fire_gathers(0, 0)

    def body(j, _):
        s = lax.rem(j, 2)
        ns = 1 - s

        @pl.when(j >= 1)
        def _():
            @pl.when(ns == 0)
            def _():
                drain_writes(0)

            @pl.when(ns == 1)
            def _():
                drain_writes(1)

        @pl.when(j + 1 < cpw)
        def _():
            @pl.when(ns == 0)
            def _():
                fire_gathers(j + 1, 0)

            @pl.when(ns == 1)
            def _():
                fire_gathers(j + 1, 1)

        @pl.when(s == 0)
        def _():
            drain_gathers(0)
            fire_writes(j, 0)

        @pl.when(s == 1)
        def _():
            drain_gathers(1)
            fire_writes(j, 1)

        return 0

    lax.fori_loop(0, cpw, body, 0)
    drain_writes((cpw - 1) % 2)


def _gather_pass(ktab, qtab, vtab, src_p, dst_p, e_base, cpw, ne):
    import functools
    mesh = plsc.VectorSubcoreMesh(core_axis_name="c", subcore_axis_name="s")
    ed = jax.ShapeDtypeStruct((ne, D), jnp.float32)
    buf = pltpu.VMEM((CHUNK, D), jnp.float32)
    fn = pl.kernel(
        functools.partial(_gather_body, cpw, e_base),
        out_type=(ed, ed, ed),
        mesh=mesh,
        scratch_types=[
            pltpu.VMEM((cpw * CHUNK,), jnp.int32),
            pltpu.VMEM((cpw * CHUNK,), jnp.int32),
            buf, buf, buf, buf, buf, buf,
            pltpu.SemaphoreType.DMA,
            pltpu.SemaphoreType.DMA,
            pltpu.SemaphoreType.DMA,
            pltpu.SemaphoreType.DMA,
        ],
    )
    return fn(ktab, qtab, vtab, src_p, dst_p)


def _edge_kernel(ks_ref, qd_ref, vs_ref, ea_ref, sbf_ref,
                 wekt_ref, wev_ref, ws0_ref, ws1_ref, msg_ref, eal_ref):
    ks = ks_ref[...]
    qd = qd_ref[...]
    ea = ea_ref[...]
    qw = jnp.dot(qd, wekt_ref[...], preferred_element_type=jnp.float32)
    alpha = (jnp.sum(ks * qd, axis=1, keepdims=True)
             + jnp.sum(qw * ea, axis=1, keepdims=True)) * _SQRT_C_INV
    ealpha = jnp.exp(alpha)
    ev = jnp.dot(ea, wev_ref[...], preferred_element_type=jnp.float32)
    sb = jnp.dot(jnp.dot(sbf_ref[...], ws0_ref[...],
                         preferred_element_type=jnp.float32),
                 ws1_ref[...], preferred_element_type=jnp.float32)
    msg_ref[...] = (vs_ref[...] + ev) * sb * ealpha
    eal_ref[...] = ealpha


def _edge_pass(ksrc, qdst, vsrc, ea_p, sbf_p, W_ekT, W_ev, W_sbf0, W_sbf1,
               e_base):
    B = 512
    ne = ksrc.shape[0]
    g = ne // B
    ob = e_base // B
    row = lambda i: (i, 0)
    rowo = lambda i: (i + ob, 0)
    full = lambda i: (0, 0)
    return pl.pallas_call(
        _edge_kernel,
        grid=(g,),
        in_specs=[
            pl.BlockSpec((B, D), row),
            pl.BlockSpec((B, D), row),
            pl.BlockSpec((B, D), row),
            pl.BlockSpec((B, ED), rowo),
            pl.BlockSpec((B, ED), rowo),
            pl.BlockSpec((D, ED), full),
            pl.BlockSpec((ED, D), full),
            pl.BlockSpec((ED, D), full),
            pl.BlockSpec((D, D), full),
        ],
        out_specs=[
            pl.BlockSpec((B, D), row),
            pl.BlockSpec((B, 1), row),
        ],
        out_shape=[
            jax.ShapeDtypeStruct((ne, D), jnp.float32),
            jax.ShapeDtypeStruct((ne, 1), jnp.float32),
        ],
    )(ksrc, qdst, vsrc, ea_p, sbf_p, W_ekT, W_ev, W_sbf0, W_sbf1)


def _scatter_body(cpw, e_base, msg_h, eal_h, dst_h, zeros_h, acc_o, den_o,
                  idx_d, mbuf, ebuf, tbuf, dacc, denom, acc, den_sh):
    cid = lax.axis_index("c")
    sid = lax.axis_index("s")
    wid = sid * NC + cid

    if True:
        def initz(s, _):
            r0 = sid * RPS + s * CHUNK
            pltpu.sync_copy(zeros_h.at[pl.ds(r0, CHUNK)],
                            acc.at[pl.ds(r0, CHUNK)])
            return 0

        def initd(i, _):
            denom[pl.ds(i * 16, 16)] = jnp.zeros((16,), jnp.float32)
            return 0

        lax.fori_loop(0, RPS // CHUNK, initz, 0)
        lax.fori_loop(0, NPAD // 16, initd, 0)
        plsc.subcore_barrier()

        def chunk(j, _):
            lbase = (wid * cpw + j) * CHUNK
            pltpu.sync_copy(dst_h.at[pl.ds(e_base + lbase, CHUNK)], idx_d)
            pltpu.sync_copy(msg_h.at[pl.ds(lbase, CHUNK)], mbuf)
            pltpu.sync_copy(eal_h.at[pl.ds(lbase, CHUNK)], ebuf)
            pltpu.sync_copy(mbuf, acc.at[idx_d], add=True)
            for k in range(CHUNK // 16):
                i16 = idx_d[pl.ds(k * 16, 16)]
                e16 = ebuf[pl.ds(k * 16, 16)]
                plsc.addupdate_scatter(denom, [i16], e16)
            return 0

        lax.fori_loop(0, cpw, chunk, 0)

        pltpu.sync_copy(denom, den_sh.at[sid])
        plsc.subcore_barrier()

        def zslice(i, _):
            dacc[pl.ds(i * 16, 16)] = jnp.zeros((16,), jnp.float32)
            return 0

        lax.fori_loop(0, RPS // 16, zslice, 0)

        def redt(t, _):
            pltpu.sync_copy(den_sh.at[t, pl.ds(sid * RPS, RPS)], tbuf)

            def addv(i, _):
                dacc[pl.ds(i * 16, 16)] = (dacc[pl.ds(i * 16, 16)]
                                           + tbuf[pl.ds(i * 16, 16)])
                return 0

            lax.fori_loop(0, RPS // 16, addv, 0)
            return 0

        lax.fori_loop(0, NS, redt, 0)

        r0 = sid * RPS
        pltpu.sync_copy(acc.at[pl.ds(r0, RPS)],
                        acc_o.at[cid, pl.ds(r0, RPS)])
        pltpu.sync_copy(dacc, den_o.at[cid, pl.ds(r0, RPS)])


def _scatter_pass(msg, eal, dst_p, zeros_nd, e_base, cpw):
    import functools
    mesh = plsc.VectorSubcoreMesh(core_axis_name="c", subcore_axis_name="s")
    fn = pl.kernel(
        functools.partial(_scatter_body, cpw, e_base),
        out_type=(
            jax.ShapeDtypeStruct((NC, NPAD, D), jnp.float32),
            jax.ShapeDtypeStruct((NC, NPAD), jnp.float32),
        ),
        mesh=mesh,
        scratch_types=[
            pltpu.VMEM((CHUNK,), jnp.int32),
            pltpu.VMEM((CHUNK, D), jnp.float32),
            pltpu.VMEM((CHUNK,), jnp.float32),
            pltpu.VMEM((RPS,), jnp.float32),
            pltpu.VMEM((RPS,), jnp.float32),
            pltpu.VMEM((NPAD,), jnp.float32),
            pltpu.VMEM_SHARED((NPAD, D), jnp.float32),
            pltpu.VMEM_SHARED((NS, NPAD), jnp.float32),
        ],
        compiler_params=pltpu.CompilerParams(needs_layout_passes=False),
    )
    return fn(msg, eal, dst_p, zeros_nd)


def _final_kernel(a0_ref, a1_ref, a2_ref, a3_ref,
                  d0_ref, d1_ref, d2_ref, d3_ref, skip_ref, out_ref):
    a = (a0_ref[0] + a1_ref[0]) + (a2_ref[0] + a3_ref[0])
    den = (d0_ref[0] + d1_ref[0]) + (d2_ref[0] + d3_ref[0]) + 1e-16
    out_ref[...] = a / den + skip_ref[...]


def _final_pass(accsA, accsB, dens3A, dens3B, skip):
    B = 512
    g = pl.cdiv(N, B)
    row = lambda i: (i, 0)
    c0 = lambda i: (0, i, 0)
    c1 = lambda i: (1, i, 0)
    return pl.pallas_call(
        _final_kernel,
        grid=(g,),
        in_specs=[
            pl.BlockSpec((1, B, D), c0),
            pl.BlockSpec((1, B, D), c1),
            pl.BlockSpec((1, B, D), c0),
            pl.BlockSpec((1, B, D), c1),
            pl.BlockSpec((1, B, 1), c0),
            pl.BlockSpec((1, B, 1), c1),
            pl.BlockSpec((1, B, 1), c0),
            pl.BlockSpec((1, B, 1), c1),
            pl.BlockSpec((B, D), row),
        ],
        out_specs=pl.BlockSpec((B, D), row),
        out_shape=jax.ShapeDtypeStruct((N, D), jnp.float32),
    )(accsA, accsA, accsB, accsB, dens3A, dens3A, dens3B, dens3B, skip)


def kernel(x, edge_index, edge_attr, rbf, sbf, edge_index_0,
           W_rbf0, W_rbf1, W_sbf0, W_sbf1, W_ek, W_ev,
           W_k, b_k, W_q, b_q, W_v, b_v, W_skip, b_skip):
    pad_e = E2 - E
    src_p = jnp.concatenate(
        [edge_index[0], jnp.zeros((pad_e,), jnp.int32)])
    dst_p = jnp.concatenate(
        [edge_index[1], jnp.full((pad_e,), N, jnp.int32)])
    ea_p = edge_attr
    sbf_p = sbf.reshape(E, ED)
    x_p = jnp.pad(x, ((0, NPAD - N), (0, 0)))
    rbf_p = jnp.pad(rbf, ((0, NPAD - N), (0, 0)))

    ktab, qtab, vtab, skip = _node_pass(
        x_p, rbf_p, W_rbf0, W_rbf1, W_k, b_k, W_q, b_q, W_v, b_v,
        W_skip, b_skip)

    CPW_A, CPW_B = 40, 39
    EA_N = NW * CPW_A * CHUNK
    EB_N = NW * CPW_B * CHUNK
    W_ekT = W_ek.T
    zeros_nd = jnp.zeros((NPAD, D), jnp.float32)

    gA = _gather_pass(ktab, qtab, vtab, src_p, dst_p, 0, CPW_A, EA_N)
    gB = _gather_pass(ktab, qtab, vtab, src_p, dst_p, EA_N, CPW_B, EB_N)

    msgA, ealA = _edge_pass(*gA, ea_p, sbf_p, W_ekT, W_ev, W_sbf0, W_sbf1, 0)
    msgB, ealB = _edge_pass(*gB, ea_p, sbf_p, W_ekT, W_ev, W_sbf0, W_sbf1,
                            EA_N)

    accsA, densA = _scatter_pass(msgA, ealA.reshape(EA_N), dst_p, zeros_nd,
                                 0, CPW_A)
    accsB, densB = _scatter_pass(msgB, ealB.reshape(EB_N), dst_p, zeros_nd,
                                 EA_N, CPW_B)

    return _final_pass(accsA, accsB,
                       densA.reshape(NC, NPAD, 1), densB.reshape(NC, NPAD, 1),
                       skip)


# speedup vs baseline: 4.7698x; 1.0067x over previous
import math

import jax
import jax.numpy as jnp
from jax import lax
from jax.experimental import pallas as pl
from jax.experimental.pallas import tpu as pltpu
from jax.experimental.pallas import tpu_sc as plsc

N = 10000
E = 320000
D = 128
C = 128
ED = 16
SBF6 = 42

NC = 2
NS = 16
NW = NC * NS
CHUNK = 128
CPW = 79
E2 = NW * CPW * CHUNK
NPAD = 10240
RPS = NPAD // NS

_SQRT_C_INV = 1.0 / math.sqrt(C)


def _node_kernel(x_ref, rbf_ref, wr0_ref, wr1_ref, wk_ref, bk_ref, wq_ref,
                 bq_ref, wv_ref, bv_ref, ws_ref, bs_ref,
                 k_ref, q_ref, v_ref, skip_ref):
    x = x_ref[...]
    rbf_f = jnp.dot(jnp.dot(rbf_ref[...], wr0_ref[...],
                            preferred_element_type=jnp.float32),
                    wr1_ref[...], preferred_element_type=jnp.float32)
    x_src = rbf_f * x
    k_ref[...] = jnp.dot(x_src, wk_ref[...],
                         preferred_element_type=jnp.float32) + bk_ref[...]
    q_ref[...] = jnp.dot(x, wq_ref[...],
                         preferred_element_type=jnp.float32) + bq_ref[...]
    v_ref[...] = jnp.dot(x_src, wv_ref[...],
                         preferred_element_type=jnp.float32) + bv_ref[...]
    skip_ref[...] = jnp.dot(x, ws_ref[...],
                            preferred_element_type=jnp.float32) + bs_ref[...]


def _node_pass(x_p, rbf_p, W_rbf0, W_rbf1, W_k, b_k, W_q, b_q, W_v, b_v,
               W_skip, b_skip):
    B = 512
    g = NPAD // B
    row = lambda i: (i, 0)
    full = lambda i: (0, 0)
    nd = jax.ShapeDtypeStruct((NPAD, D), jnp.float32)
    return pl.pallas_call(
        _node_kernel,
        grid=(g,),
        in_specs=[
            pl.BlockSpec((B, D), row),
            pl.BlockSpec((B, SBF6), row),
            pl.BlockSpec((SBF6, D), full),
            pl.BlockSpec((D, D), full),
            pl.BlockSpec((D, D), full),
            pl.BlockSpec((1, D), full),
            pl.BlockSpec((D, D), full),
            pl.BlockSpec((1, D), full),
            pl.BlockSpec((D, D), full),
            pl.BlockSpec((1, D), full),
            pl.BlockSpec((D, D), full),
            pl.BlockSpec((1, D), full),
        ],
        out_specs=[pl.BlockSpec((B, D), row)] * 4,
        out_shape=[nd, nd, nd, nd],
    )(x_p, rbf_p, W_rbf0, W_rbf1, W_k, b_k.reshape(1, D), W_q,
      b_q.reshape(1, D), W_v, b_v.reshape(1, D), W_skip,
      b_skip.reshape(1, D))


def _gather_body(cpw, e_base, ktab, qtab, vtab, src_h, dst_h,
                 ksrc_o, qdst_o, vsrc_o,
                 idx_s, idx_d, bk0, bq0, bv0, bk1, bq1, bv1,
                 gs0, gs1, ws0, ws1):
    epw = cpw * CHUNK
    wid = lax.axis_index("s") * NC + lax.axis_index("c")
    e0 = e_base + wid * epw
    pltpu.sync_copy(src_h.at[pl.ds(e0, epw)], idx_s)
    pltpu.sync_copy(dst_h.at[pl.ds(e0, epw)], idx_d)

    bufs = ((bk0, bq0, bv0), (bk1, bq1, bv1))
    gsems = (gs0, gs1)
    wsems = (ws0, ws1)

    def fire_gathers(j, s):
        o = j * CHUNK
        i_s = idx_s.at[pl.ds(o, CHUNK)]
        i_d = idx_d.at[pl.ds(o, CHUNK)]
        pltpu.async_copy(ktab.at[i_s], bufs[s][0], gsems[s])
        pltpu.async_copy(qtab.at[i_d], bufs[s][1], gsems[s])
        pltpu.async_copy(vtab.at[i_s], bufs[s][2], gsems[s])

    def drain_gathers(s):
        for b in bufs[s]:
            pltpu.make_async_copy(ktab.at[idx_s.at[pl.ds(0, CHUNK)]],
                                  b, gsems[s]).wait()

    def fire_writes(j, s):
        base = wid * epw + j * CHUNK
        pltpu.async_copy(bufs[s][0], ksrc_o.at[pl.ds(base, CHUNK)], wsems[s])
        pltpu.async_copy(bufs[s][1], qdst_o.at[pl.ds(base, CHUNK)], wsems[s])
        pltpu.async_copy(bufs[s][2], vsrc_o.at[pl.ds(base, CHUNK)], wsems[s])

    def drain_writes(s):
        for b in bufs[s]:
            pltpu.make_async_copy(b, ksrc_o.at[pl.ds(0, CHUNK)],
                                  wsems[s]).wait()

    fire_gathers(0, 0)

    def body(j, _):
        s = lax.rem(j, 2)
        ns = 1 - s

        @pl.when(j >= 1)
        def _():
            @pl.when(ns == 0)
            def _():
                drain_writes(0)

            @pl.when(ns == 1)
            def _():
                drain_writes(1)

        @pl.when(j + 1 < cpw)
        def _():
            @pl.when(ns == 0)
            def _():
                fire_gathers(j + 1, 0)

            @pl.when(ns == 1)
            def _():
                fire_gathers(j + 1, 1)

        @pl.when(s == 0)
        def _():
            drain_gathers(0)
            fire_writes(j, 0)

        @pl.when(s == 1)
        def _():
            drain_gathers(1)
            fire_writes(j, 1)

        return 0

    lax.fori_loop(0, cpw, body, 0)
    drain_writes((cpw - 1) % 2)


def _gather_pass(ktab, qtab, vtab, src_p, dst_p, e_base, cpw, ne):
    import functools
    mesh = plsc.VectorSubcoreMesh(core_axis_name="c", subcore_axis_name="s")
    ed = jax.ShapeDtypeStruct((ne, D), jnp.float32)
    buf = pltpu.VMEM((CHUNK, D), jnp.float32)
    fn = pl.kernel(
        functools.partial(_gather_body, cpw, e_base),
        out_type=(ed, ed, ed),
        mesh=mesh,
        scratch_types=[
            pltpu.VMEM((cpw * CHUNK,), jnp.int32),
            pltpu.VMEM((cpw * CHUNK,), jnp.int32),
            buf, buf, buf, buf, buf, buf,
            pltpu.SemaphoreType.DMA,
            pltpu.SemaphoreType.DMA,
            pltpu.SemaphoreType.DMA,
            pltpu.SemaphoreType.DMA,
        ],
    )
    return fn(ktab, qtab, vtab, src_p, dst_p)


def _edge_kernel(ks_ref, qd_ref, vs_ref, ea_ref, sbf_ref,
                 wekt_ref, wev_ref, ws0_ref, ws1_ref, msg_ref, eal_ref):
    ks = ks_ref[...]
    qd = qd_ref[...]
    ea = ea_ref[...]
    qw = jnp.dot(qd, wekt_ref[...], preferred_element_type=jnp.float32)
    alpha = (jnp.sum(ks * qd, axis=1, keepdims=True)
             + jnp.sum(qw * ea, axis=1, keepdims=True)) * _SQRT_C_INV
    ealpha = jnp.exp(alpha)
    ev = jnp.dot(ea, wev_ref[...], preferred_element_type=jnp.float32)
    sb = jnp.dot(jnp.dot(sbf_ref[...], ws0_ref[...],
                         preferred_element_type=jnp.float32),
                 ws1_ref[...], preferred_element_type=jnp.float32)
    msg_ref[...] = (vs_ref[...] + ev) * sb * ealpha
    eal_ref[...] = ealpha


def _edge_pass(ksrc, qdst, vsrc, ea_p, sbf_p, W_ekT, W_ev, W_sbf0, W_sbf1,
               e_base):
    B = 512
    ne = ksrc.shape[0]
    g = ne // B
    ob = e_base // B
    row = lambda i: (i, 0)
    rowo = lambda i: (i + ob, 0)
    full = lambda i: (0, 0)
    return pl.pallas_call(
        _edge_kernel,
        grid=(g,),
        in_specs=[
            pl.BlockSpec((B, D), row),
            pl.BlockSpec((B, D), row),
            pl.BlockSpec((B, D), row),
            pl.BlockSpec((B, ED), rowo),
            pl.BlockSpec((B, ED), rowo),
            pl.BlockSpec((D, ED), full),
            pl.BlockSpec((ED, D), full),
            pl.BlockSpec((ED, D), full),
            pl.BlockSpec((D, D), full),
        ],
        out_specs=[
            pl.BlockSpec((B, D), row),
            pl.BlockSpec((B, 1), row),
        ],
        out_shape=[
            jax.ShapeDtypeStruct((ne, D), jnp.float32),
            jax.ShapeDtypeStruct((ne, 1), jnp.float32),
        ],
    )(ksrc, qdst, vsrc, ea_p, sbf_p, W_ekT, W_ev, W_sbf0, W_sbf1)


def _scatter_body(cpw, e_base, msg_h, eal_h, dst_h, acc_o, den_o,
                  idx_d, mbuf, ebuf, tbuf, dacc, denom, acc, den_sh):
    cid = lax.axis_index("c")
    sid = lax.axis_index("s")
    wid = sid * NC + cid

    if True:
        def zrow(r, _):
            for k in range(D // 16):
                mbuf[r, pl.ds(k * 16, 16)] = jnp.zeros((16,), jnp.float32)
            return 0

        def initz(s, _):
            r0 = sid * RPS + s * CHUNK
            pltpu.sync_copy(mbuf, acc.at[pl.ds(r0, CHUNK)])
            return 0

        def initd(i, _):
            denom[pl.ds(i * 16, 16)] = jnp.zeros((16,), jnp.float32)
            return 0

        lax.fori_loop(0, CHUNK, zrow, 0)
        lax.fori_loop(0, RPS // CHUNK, initz, 0)
        lax.fori_loop(0, NPAD // 16, initd, 0)
        plsc.subcore_barrier()

        def chunk(j, _):
            lbase = (wid * cpw + j) * CHUNK
            pltpu.sync_copy(dst_h.at[pl.ds(e_base + lbase, CHUNK)], idx_d)
            pltpu.sync_copy(msg_h.at[pl.ds(lbase, CHUNK)], mbuf)
            pltpu.sync_copy(eal_h.at[pl.ds(lbase, CHUNK)], ebuf)
            pltpu.sync_copy(mbuf, acc.at[idx_d], add=True)
            for k in range(CHUNK // 16):
                i16 = idx_d[pl.ds(k * 16, 16)]
                e16 = ebuf[pl.ds(k * 16, 16)]
                plsc.addupdate_scatter(denom, [i16], e16)
            return 0

        lax.fori_loop(0, cpw, chunk, 0)

        pltpu.sync_copy(denom, den_sh.at[sid])
        plsc.subcore_barrier()

        def zslice(i, _):
            dacc[pl.ds(i * 16, 16)] = jnp.zeros((16,), jnp.float32)
            return 0

        lax.fori_loop(0, RPS // 16, zslice, 0)

        def redt(t, _):
            pltpu.sync_copy(den_sh.at[t, pl.ds(sid * RPS, RPS)], tbuf)

            def addv(i, _):
                dacc[pl.ds(i * 16, 16)] = (dacc[pl.ds(i * 16, 16)]
                                           + tbuf[pl.ds(i * 16, 16)])
                return 0

            lax.fori_loop(0, RPS // 16, addv, 0)
            return 0

        lax.fori_loop(0, NS, redt, 0)

        r0 = sid * RPS
        pltpu.sync_copy(acc.at[pl.ds(r0, RPS)],
                        acc_o.at[cid, pl.ds(r0, RPS)])
        pltpu.sync_copy(dacc, den_o.at[cid, pl.ds(r0, RPS)])


def _scatter_pass(msg, eal, dst_p, e_base, cpw):
    import functools
    mesh = plsc.VectorSubcoreMesh(core_axis_name="c", subcore_axis_name="s")
    fn = pl.kernel(
        functools.partial(_scatter_body, cpw, e_base),
        out_type=(
            jax.ShapeDtypeStruct((NC, NPAD, D), jnp.float32),
            jax.ShapeDtypeStruct((NC, NPAD), jnp.float32),
        ),
        mesh=mesh,
        scratch_types=[
            pltpu.VMEM((CHUNK,), jnp.int32),
            pltpu.VMEM((CHUNK, D), jnp.float32),
            pltpu.VMEM((CHUNK,), jnp.float32),
            pltpu.VMEM((RPS,), jnp.float32),
            pltpu.VMEM((RPS,), jnp.float32),
            pltpu.VMEM((NPAD,), jnp.float32),
            pltpu.VMEM_SHARED((NPAD, D), jnp.float32),
            pltpu.VMEM_SHARED((NS, NPAD), jnp.float32),
        ],
        compiler_params=pltpu.CompilerParams(needs_layout_passes=False),
    )
    return fn(msg, eal, dst_p)


def _final_kernel(a0_ref, a1_ref, a2_ref, a3_ref,
                  d0_ref, d1_ref, d2_ref, d3_ref, skip_ref, out_ref):
    a = (a0_ref[0] + a1_ref[0]) + (a2_ref[0] + a3_ref[0])
    den = (d0_ref[0] + d1_ref[0]) + (d2_ref[0] + d3_ref[0]) + 1e-16
    out_ref[...] = a / den + skip_ref[...]


def _final_pass(accsA, accsB, dens3A, dens3B, skip):
    B = 512
    g = pl.cdiv(N, B)
    row = lambda i: (i, 0)
    c0 = lambda i: (0, i, 0)
    c1 = lambda i: (1, i, 0)
    return pl.pallas_call(
        _final_kernel,
        grid=(g,),
        in_specs=[
            pl.BlockSpec((1, B, D), c0),
            pl.BlockSpec((1, B, D), c1),
            pl.BlockSpec((1, B, D), c0),
            pl.BlockSpec((1, B, D), c1),
            pl.BlockSpec((1, B, 1), c0),
            pl.BlockSpec((1, B, 1), c1),
            pl.BlockSpec((1, B, 1), c0),
            pl.BlockSpec((1, B, 1), c1),
            pl.BlockSpec((B, D), row),
        ],
        out_specs=pl.BlockSpec((B, D), row),
        out_shape=jax.ShapeDtypeStruct((N, D), jnp.float32),
    )(accsA, accsA, accsB, accsB, dens3A, dens3A, dens3B, dens3B, skip)


def kernel(x, edge_index, edge_attr, rbf, sbf, edge_index_0,
           W_rbf0, W_rbf1, W_sbf0, W_sbf1, W_ek, W_ev,
           W_k, b_k, W_q, b_q, W_v, b_v, W_skip, b_skip):
    pad_e = E2 - E
    src_p = jnp.concatenate(
        [edge_index[0], jnp.zeros((pad_e,), jnp.int32)])
    dst_p = jnp.concatenate(
        [edge_index[1], jnp.full((pad_e,), N, jnp.int32)])
    ea_p = edge_attr
    sbf_p = sbf.reshape(E, ED)
    x_p = jnp.pad(x, ((0, NPAD - N), (0, 0)))
    rbf_p = jnp.pad(rbf, ((0, NPAD - N), (0, 0)))

    ktab, qtab, vtab, skip = _node_pass(
        x_p, rbf_p, W_rbf0, W_rbf1, W_k, b_k, W_q, b_q, W_v, b_v,
        W_skip, b_skip)

    CPW_A, CPW_B = 40, 39
    EA_N = NW * CPW_A * CHUNK
    EB_N = NW * CPW_B * CHUNK
    W_ekT = W_ek.T

    gA = _gather_pass(ktab, qtab, vtab, src_p, dst_p, 0, CPW_A, EA_N)
    gB = _gather_pass(ktab, qtab, vtab, src_p, dst_p, EA_N, CPW_B, EB_N)

    msgA, ealA = _edge_pass(*gA, ea_p, sbf_p, W_ekT, W_ev, W_sbf0, W_sbf1, 0)
    msgB, ealB = _edge_pass(*gB, ea_p, sbf_p, W_ekT, W_ev, W_sbf0, W_sbf1,
                            EA_N)

    accsA, densA = _scatter_pass(msgA, ealA.reshape(EA_N), dst_p, 0, CPW_A)
    accsB, densB = _scatter_pass(msgB, ealB.reshape(EB_N), dst_p, EA_N, CPW_B)

    return _final_pass(accsA, accsB,
                       densA.reshape(NC, NPAD, 1), densB.reshape(NC, NPAD, 1),
                       skip)


# speedup vs baseline: 4.7756x; 1.0012x over previous
import math

import jax
import jax.numpy as jnp
from jax import lax
from jax.experimental import pallas as pl
from jax.experimental.pallas import tpu as pltpu
from jax.experimental.pallas import tpu_sc as plsc

N = 10000
E = 320000
D = 128
C = 128
ED = 16
SBF6 = 42

NC = 2
NS = 16
NW = NC * NS
CHUNK = 128
CPW = 79
E2 = NW * CPW * CHUNK
NPAD = 10240
RPS = NPAD // NS

_SQRT_C_INV = 1.0 / math.sqrt(C)


def _node_kernel(x_ref, rbf_ref, wr0_ref, wr1_ref, wk_ref, bk_ref, wq_ref,
                 bq_ref, wv_ref, bv_ref, ws_ref, bs_ref,
                 kv_ref, q_ref, skip_ref):
    x = x_ref[...]
    rbf_f = jnp.dot(jnp.dot(rbf_ref[...], wr0_ref[...],
                            preferred_element_type=jnp.float32),
                    wr1_ref[...], preferred_element_type=jnp.float32)
    x_src = rbf_f * x
    kv_ref[:, :D] = jnp.dot(x_src, wk_ref[...],
                            preferred_element_type=jnp.float32) + bk_ref[...]
    kv_ref[:, D:] = jnp.dot(x_src, wv_ref[...],
                            preferred_element_type=jnp.float32) + bv_ref[...]
    q_ref[...] = jnp.dot(x, wq_ref[...],
                         preferred_element_type=jnp.float32) + bq_ref[...]
    skip_ref[...] = jnp.dot(x, ws_ref[...],
                            preferred_element_type=jnp.float32) + bs_ref[...]


def _node_pass(x_p, rbf_p, W_rbf0, W_rbf1, W_k, b_k, W_q, b_q, W_v, b_v,
               W_skip, b_skip):
    B = 512
    g = NPAD // B
    row = lambda i: (i, 0)
    full = lambda i: (0, 0)
    nd = jax.ShapeDtypeStruct((NPAD, D), jnp.float32)
    return pl.pallas_call(
        _node_kernel,
        grid=(g,),
        in_specs=[
            pl.BlockSpec((B, D), row),
            pl.BlockSpec((B, SBF6), row),
            pl.BlockSpec((SBF6, D), full),
            pl.BlockSpec((D, D), full),
            pl.BlockSpec((D, D), full),
            pl.BlockSpec((1, D), full),
            pl.BlockSpec((D, D), full),
            pl.BlockSpec((1, D), full),
            pl.BlockSpec((D, D), full),
            pl.BlockSpec((1, D), full),
            pl.BlockSpec((D, D), full),
            pl.BlockSpec((1, D), full),
        ],
        out_specs=[pl.BlockSpec((B, 2 * D), row),
                   pl.BlockSpec((B, D), row),
                   pl.BlockSpec((B, D), row)],
        out_shape=[jax.ShapeDtypeStruct((NPAD, 2 * D), jnp.float32), nd, nd],
    )(x_p, rbf_p, W_rbf0, W_rbf1, W_k, b_k.reshape(1, D), W_q,
      b_q.reshape(1, D), W_v, b_v.reshape(1, D), W_skip,
      b_skip.reshape(1, D))


def _gather_body(cpw, e_base, kvtab, qtab, src_h, dst_h,
                 kvsrc_o, qdst_o,
                 idx_s, idx_d, bkv0, bq0, bkv1, bq1,
                 gs0, gs1, ws0, ws1):
    epw = cpw * CHUNK
    wid = lax.axis_index("s") * NC + lax.axis_index("c")
    e0 = e_base + wid * epw
    pltpu.sync_copy(src_h.at[pl.ds(e0, epw)], idx_s)
    pltpu.sync_copy(dst_h.at[pl.ds(e0, epw)], idx_d)

    bufs = ((bkv0, bq0), (bkv1, bq1))
    gsems = (gs0, gs1)
    wsems = (ws0, ws1)

    def fire_gathers(j, s):
        o = j * CHUNK
        i_s = idx_s.at[pl.ds(o, CHUNK)]
        i_d = idx_d.at[pl.ds(o, CHUNK)]
        pltpu.async_copy(kvtab.at[i_s], bufs[s][0], gsems[s])
        pltpu.async_copy(qtab.at[i_d], bufs[s][1], gsems[s])

    def drain_gathers(s):
        pltpu.make_async_copy(kvtab.at[idx_s.at[pl.ds(0, CHUNK)]],
                              bufs[s][0], gsems[s]).wait()
        pltpu.make_async_copy(qtab.at[idx_d.at[pl.ds(0, CHUNK)]],
                              bufs[s][1], gsems[s]).wait()

    def fire_writes(j, s):
        base = wid * epw + j * CHUNK
        pltpu.async_copy(bufs[s][0], kvsrc_o.at[pl.ds(base, CHUNK)], wsems[s])
        pltpu.async_copy(bufs[s][1], qdst_o.at[pl.ds(base, CHUNK)], wsems[s])

    def drain_writes(s):
        pltpu.make_async_copy(bufs[s][0], kvsrc_o.at[pl.ds(0, CHUNK)],
                              wsems[s]).wait()
        pltpu.make_async_copy(bufs[s][1], qdst_o.at[pl.ds(0, CHUNK)],
                              wsems[s]).wait()

    fire_gathers(0, 0)

    def body(j, _):
        s = lax.rem(j, 2)
        ns = 1 - s

        @pl.when(j >= 1)
        def _():
            @pl.when(ns == 0)
            def _():
                drain_writes(0)

            @pl.when(ns == 1)
            def _():
                drain_writes(1)

        @pl.when(j + 1 < cpw)
        def _():
            @pl.when(ns == 0)
            def _():
                fire_gathers(j + 1, 0)

            @pl.when(ns == 1)
            def _():
                fire_gathers(j + 1, 1)

        @pl.when(s == 0)
        def _():
            drain_gathers(0)
            fire_writes(j, 0)

        @pl.when(s == 1)
        def _():
            drain_gathers(1)
            fire_writes(j, 1)

        return 0

    lax.fori_loop(0, cpw, body, 0)
    drain_writes((cpw - 1) % 2)


def _gather_pass(kvtab, qtab, src_p, dst_p, e_base, cpw, ne):
    import functools
    mesh = plsc.VectorSubcoreMesh(core_axis_name="c", subcore_axis_name="s")
    bkv = pltpu.VMEM((CHUNK, 2 * D), jnp.float32)
    bq = pltpu.VMEM((CHUNK, D), jnp.float32)
    fn = pl.kernel(
        functools.partial(_gather_body, cpw, e_base),
        out_type=(jax.ShapeDtypeStruct((ne, 2 * D), jnp.float32),
                  jax.ShapeDtypeStruct((ne, D), jnp.float32)),
        mesh=mesh,
        scratch_types=[
            pltpu.VMEM((cpw * CHUNK,), jnp.int32),
            pltpu.VMEM((cpw * CHUNK,), jnp.int32),
            bkv, bq, bkv, bq,
            pltpu.SemaphoreType.DMA,
            pltpu.SemaphoreType.DMA,
            pltpu.SemaphoreType.DMA,
            pltpu.SemaphoreType.DMA,
        ],
    )
    return fn(kvtab, qtab, src_p, dst_p)


def _edge_kernel(kvs_ref, qd_ref, ea_ref, sbf_ref,
                 wekt_ref, wev_ref, ws0_ref, ws1_ref, msg_ref, eal_ref):
    ks = kvs_ref[:, :D]
    vs = kvs_ref[:, D:]
    qd = qd_ref[...]
    ea = ea_ref[...]
    qw = jnp.dot(qd, wekt_ref[...], preferred_element_type=jnp.float32)
    alpha = (jnp.sum(ks * qd, axis=1, keepdims=True)
             + jnp.sum(qw * ea, axis=1, keepdims=True)) * _SQRT_C_INV
    ealpha = jnp.exp(alpha)
    ev = jnp.dot(ea, wev_ref[...], preferred_element_type=jnp.float32)
    sb = jnp.dot(jnp.dot(sbf_ref[...], ws0_ref[...],
                         preferred_element_type=jnp.float32),
                 ws1_ref[...], preferred_element_type=jnp.float32)
    msg_ref[...] = (vs + ev) * sb * ealpha
    eal_ref[...] = ealpha


def _edge_pass(kvsrc, qdst, ea_p, sbf_p, W_ekT, W_ev, W_sbf0, W_sbf1,
               e_base):
    B = 512
    ne = kvsrc.shape[0]
    g = ne // B
    ob = e_base // B
    row = lambda i: (i, 0)
    rowo = lambda i: (i + ob, 0)
    full = lambda i: (0, 0)
    return pl.pallas_call(
        _edge_kernel,
        grid=(g,),
        in_specs=[
            pl.BlockSpec((B, 2 * D), row),
            pl.BlockSpec((B, D), row),
            pl.BlockSpec((B, ED), rowo),
            pl.BlockSpec((B, ED), rowo),
            pl.BlockSpec((D, ED), full),
            pl.BlockSpec((ED, D), full),
            pl.BlockSpec((ED, D), full),
            pl.BlockSpec((D, D), full),
        ],
        out_specs=[
            pl.BlockSpec((B, D), row),
            pl.BlockSpec((B, 1), row),
        ],
        out_shape=[
            jax.ShapeDtypeStruct((ne, D), jnp.float32),
            jax.ShapeDtypeStruct((ne, 1), jnp.float32),
        ],
    )(kvsrc, qdst, ea_p, sbf_p, W_ekT, W_ev, W_sbf0, W_sbf1)


def _scatter_body(cpw, e_base, msg_h, eal_h, dst_h, acc_o, den_o,
                  idx_d, mbuf, ebuf, tbuf, dacc, denom, acc, den_sh):
    cid = lax.axis_index("c")
    sid = lax.axis_index("s")
    wid = sid * NC + cid

    if True:
        def zrow(r, _):
            for k in range(D // 16):
                mbuf[r, pl.ds(k * 16, 16)] = jnp.zeros((16,), jnp.float32)
            return 0

        def initz(s, _):
            r0 = sid * RPS + s * CHUNK
            pltpu.sync_copy(mbuf, acc.at[pl.ds(r0, CHUNK)])
            return 0

        def initd(i, _):
            denom[pl.ds(i * 16, 16)] = jnp.zeros((16,), jnp.float32)
            return 0

        lax.fori_loop(0, CHUNK, zrow, 0)
        lax.fori_loop(0, RPS // CHUNK, initz, 0)
        lax.fori_loop(0, NPAD // 16, initd, 0)
        plsc.subcore_barrier()

        def chunk(j, _):
            lbase = (wid * cpw + j) * CHUNK
            pltpu.sync_copy(dst_h.at[pl.ds(e_base + lbase, CHUNK)], idx_d)
            pltpu.sync_copy(msg_h.at[pl.ds(lbase, CHUNK)], mbuf)
            pltpu.sync_copy(eal_h.at[pl.ds(lbase, CHUNK)], ebuf)
            pltpu.sync_copy(mbuf, acc.at[idx_d], add=True)
            for k in range(CHUNK // 16):
                i16 = idx_d[pl.ds(k * 16, 16)]
                e16 = ebuf[pl.ds(k * 16, 16)]
                plsc.addupdate_scatter(denom, [i16], e16)
            return 0

        lax.fori_loop(0, cpw, chunk, 0)

        pltpu.sync_copy(denom, den_sh.at[sid])
        plsc.subcore_barrier()

        def zslice(i, _):
            dacc[pl.ds(i * 16, 16)] = jnp.zeros((16,), jnp.float32)
            return 0

        lax.fori_loop(0, RPS // 16, zslice, 0)

        def redt(t, _):
            pltpu.sync_copy(den_sh.at[t, pl.ds(sid * RPS, RPS)], tbuf)

            def addv(i, _):
                dacc[pl.ds(i * 16, 16)] = (dacc[pl.ds(i * 16, 16)]
                                           + tbuf[pl.ds(i * 16, 16)])
                return 0

            lax.fori_loop(0, RPS // 16, addv, 0)
            return 0

        lax.fori_loop(0, NS, redt, 0)

        r0 = sid * RPS
        pltpu.sync_copy(acc.at[pl.ds(r0, RPS)],
                        acc_o.at[cid, pl.ds(r0, RPS)])
        pltpu.sync_copy(dacc, den_o.at[cid, pl.ds(r0, RPS)])


def _scatter_pass(msg, eal, dst_p, e_base, cpw):
    import functools
    mesh = plsc.VectorSubcoreMesh(core_axis_name="c", subcore_axis_name="s")
    fn = pl.kernel(
        functools.partial(_scatter_body, cpw, e_base),
        out_type=(
            jax.ShapeDtypeStruct((NC, NPAD, D), jnp.float32),
            jax.ShapeDtypeStruct((NC, NPAD), jnp.float32),
        ),
        mesh=mesh,
        scratch_types=[
            pltpu.VMEM((CHUNK,), jnp.int32),
            pltpu.VMEM((CHUNK, D), jnp.float32),
            pltpu.VMEM((CHUNK,), jnp.float32),
            pltpu.VMEM((RPS,), jnp.float32),
            pltpu.VMEM((RPS,), jnp.float32),
            pltpu.VMEM((NPAD,), jnp.float32),
            pltpu.VMEM_SHARED((NPAD, D), jnp.float32),
            pltpu.VMEM_SHARED((NS, NPAD), jnp.float32),
        ],
        compiler_params=pltpu.CompilerParams(needs_layout_passes=False),
    )
    return fn(msg, eal, dst_p)


def _final_kernel(a0_ref, a1_ref, a2_ref, a3_ref,
                  d0_ref, d1_ref, d2_ref, d3_ref, skip_ref, out_ref):
    a = (a0_ref[0] + a1_ref[0]) + (a2_ref[0] + a3_ref[0])
    den = (d0_ref[0] + d1_ref[0]) + (d2_ref[0] + d3_ref[0]) + 1e-16
    out_ref[...] = a / den + skip_ref[...]


def _final_pass(accsA, accsB, dens3A, dens3B, skip):
    B = 512
    g = pl.cdiv(N, B)
    row = lambda i: (i, 0)
    c0 = lambda i: (0, i, 0)
    c1 = lambda i: (1, i, 0)
    return pl.pallas_call(
        _final_kernel,
        grid=(g,),
        in_specs=[
            pl.BlockSpec((1, B, D), c0),
            pl.BlockSpec((1, B, D), c1),
            pl.BlockSpec((1, B, D), c0),
            pl.BlockSpec((1, B, D), c1),
            pl.BlockSpec((1, B, 1), c0),
            pl.BlockSpec((1, B, 1), c1),
            pl.BlockSpec((1, B, 1), c0),
            pl.BlockSpec((1, B, 1), c1),
            pl.BlockSpec((B, D), row),
        ],
        out_specs=pl.BlockSpec((B, D), row),
        out_shape=jax.ShapeDtypeStruct((N, D), jnp.float32),
    )(accsA, accsA, accsB, accsB, dens3A, dens3A, dens3B, dens3B, skip)


def kernel(x, edge_index, edge_attr, rbf, sbf, edge_index_0,
           W_rbf0, W_rbf1, W_sbf0, W_sbf1, W_ek, W_ev,
           W_k, b_k, W_q, b_q, W_v, b_v, W_skip, b_skip):
    pad_e = E2 - E
    src_p = jnp.concatenate(
        [edge_index[0], jnp.zeros((pad_e,), jnp.int32)])
    dst_p = jnp.concatenate(
        [edge_index[1], jnp.full((pad_e,), N, jnp.int32)])
    ea_p = edge_attr
    sbf_p = sbf.reshape(E, ED)
    x_p = jnp.pad(x, ((0, NPAD - N), (0, 0)))
    rbf_p = jnp.pad(rbf, ((0, NPAD - N), (0, 0)))

    kvtab, qtab, skip = _node_pass(
        x_p, rbf_p, W_rbf0, W_rbf1, W_k, b_k, W_q, b_q, W_v, b_v,
        W_skip, b_skip)

    CPW_A, CPW_B = 40, 39
    EA_N = NW * CPW_A * CHUNK
    EB_N = NW * CPW_B * CHUNK
    W_ekT = W_ek.T

    gA = _gather_pass(kvtab, qtab, src_p, dst_p, 0, CPW_A, EA_N)
    gB = _gather_pass(kvtab, qtab, src_p, dst_p, EA_N, CPW_B, EB_N)

    msgA, ealA = _edge_pass(*gA, ea_p, sbf_p, W_ekT, W_ev, W_sbf0, W_sbf1, 0)
    msgB, ealB = _edge_pass(*gB, ea_p, sbf_p, W_ekT, W_ev, W_sbf0, W_sbf1,
                            EA_N)

    accsA, densA = _scatter_pass(msgA, ealA.reshape(EA_N), dst_p, 0, CPW_A)
    accsB, densB = _scatter_pass(msgB, ealB.reshape(EB_N), dst_p, EA_N, CPW_B)

    return _final_pass(accsA, accsB,
                       densA.reshape(NC, NPAD, 1), densB.reshape(NC, NPAD, 1),
                       skip)


# speedup vs baseline: 4.9663x; 1.0399x over previous
import math

import jax
import jax.numpy as jnp
from jax import lax
from jax.experimental import pallas as pl
from jax.experimental.pallas import tpu as pltpu
from jax.experimental.pallas import tpu_sc as plsc

N = 10000
E = 320000
D = 128
C = 128
ED = 16
SBF6 = 42

NC = 2
NS = 16
NW = NC * NS
CHUNK = 128
CPW = 79
E2 = NW * CPW * CHUNK
NPAD = 10240
RPS = NPAD // NS
SCH = 64

_SQRT_C_INV = 1.0 / math.sqrt(C)


def _node_kernel(x_ref, rbf_ref, wr0_ref, wr1_ref, wk_ref, bk_ref, wq_ref,
                 bq_ref, wv_ref, bv_ref, ws_ref, bs_ref,
                 kv_ref, q_ref, skip_ref):
    x = x_ref[...]
    rbf_f = jnp.dot(jnp.dot(rbf_ref[...], wr0_ref[...],
                            preferred_element_type=jnp.float32),
                    wr1_ref[...], preferred_element_type=jnp.float32)
    x_src = rbf_f * x
    kv_ref[:, :D] = jnp.dot(x_src, wk_ref[...],
                            preferred_element_type=jnp.float32) + bk_ref[...]
    kv_ref[:, D:] = jnp.dot(x_src, wv_ref[...],
                            preferred_element_type=jnp.float32) + bv_ref[...]
    q_ref[...] = jnp.dot(x, wq_ref[...],
                         preferred_element_type=jnp.float32) + bq_ref[...]
    skip_ref[...] = jnp.dot(x, ws_ref[...],
                            preferred_element_type=jnp.float32) + bs_ref[...]


def _node_pass(x_p, rbf_p, W_rbf0, W_rbf1, W_k, b_k, W_q, b_q, W_v, b_v,
               W_skip, b_skip):
    B = 512
    g = NPAD // B
    row = lambda i: (i, 0)
    full = lambda i: (0, 0)
    nd = jax.ShapeDtypeStruct((NPAD, D), jnp.float32)
    return pl.pallas_call(
        _node_kernel,
        grid=(g,),
        in_specs=[
            pl.BlockSpec((B, D), row),
            pl.BlockSpec((B, SBF6), row),
            pl.BlockSpec((SBF6, D), full),
            pl.BlockSpec((D, D), full),
            pl.BlockSpec((D, D), full),
            pl.BlockSpec((1, D), full),
            pl.BlockSpec((D, D), full),
            pl.BlockSpec((1, D), full),
            pl.BlockSpec((D, D), full),
            pl.BlockSpec((1, D), full),
            pl.BlockSpec((D, D), full),
            pl.BlockSpec((1, D), full),
        ],
        out_specs=[pl.BlockSpec((B, 2 * D), row),
                   pl.BlockSpec((B, D), row),
                   pl.BlockSpec((B, D), row)],
        out_shape=[jax.ShapeDtypeStruct((NPAD, 2 * D), jnp.float32), nd, nd],
    )(x_p, rbf_p, W_rbf0, W_rbf1, W_k, b_k.reshape(1, D), W_q,
      b_q.reshape(1, D), W_v, b_v.reshape(1, D), W_skip,
      b_skip.reshape(1, D))


def _gather_body(cpw, e_base, kvtab, qtab, src_h, dst_h,
                 kvsrc_o, qdst_o,
                 idx_s, idx_d, bkv0, bq0, bkv1, bq1,
                 gs0, gs1, ws0, ws1):
    epw = cpw * CHUNK
    wid = lax.axis_index("s") * NC + lax.axis_index("c")
    e0 = e_base + wid * epw
    pltpu.sync_copy(src_h.at[pl.ds(e0, epw)], idx_s)
    pltpu.sync_copy(dst_h.at[pl.ds(e0, epw)], idx_d)

    bufs = ((bkv0, bq0), (bkv1, bq1))
    gsems = (gs0, gs1)
    wsems = (ws0, ws1)

    def fire_gathers(j, s):
        o = j * CHUNK
        i_s = idx_s.at[pl.ds(o, CHUNK)]
        i_d = idx_d.at[pl.ds(o, CHUNK)]
        pltpu.async_copy(kvtab.at[i_s], bufs[s][0], gsems[s])
        pltpu.async_copy(qtab.at[i_d], bufs[s][1], gsems[s])

    def drain_gathers(s):
        pltpu.make_async_copy(kvtab.at[idx_s.at[pl.ds(0, CHUNK)]],
                              bufs[s][0], gsems[s]).wait()
        pltpu.make_async_copy(qtab.at[idx_d.at[pl.ds(0, CHUNK)]],
                              bufs[s][1], gsems[s]).wait()

    def fire_writes(j, s):
        base = wid * epw + j * CHUNK
        pltpu.async_copy(bufs[s][0], kvsrc_o.at[pl.ds(base, CHUNK)], wsems[s])
        pltpu.async_copy(bufs[s][1], qdst_o.at[pl.ds(base, CHUNK)], wsems[s])

    def drain_writes(s):
        pltpu.make_async_copy(bufs[s][0], kvsrc_o.at[pl.ds(0, CHUNK)],
                              wsems[s]).wait()
        pltpu.make_async_copy(bufs[s][1], qdst_o.at[pl.ds(0, CHUNK)],
                              wsems[s]).wait()

    fire_gathers(0, 0)

    def body(j, _):
        s = lax.rem(j, 2)
        ns = 1 - s

        @pl.when(j >= 1)
        def _():
            @pl.when(ns == 0)
            def _():
                drain_writes(0)

            @pl.when(ns == 1)
            def _():
                drain_writes(1)

        @pl.when(j + 1 < cpw)
        def _():
            @pl.when(ns == 0)
            def _():
                fire_gathers(j + 1, 0)

            @pl.when(ns == 1)
            def _():
                fire_gathers(j + 1, 1)

        @pl.when(s == 0)
        def _():
            drain_gathers(0)
            fire_writes(j, 0)

        @pl.when(s == 1)
        def _():
            drain_gathers(1)
            fire_writes(j, 1)

        return 0

    lax.fori_loop(0, cpw, body, 0)
    drain_writes((cpw - 1) % 2)


def _gather_pass(kvtab, qtab, src_p, dst_p, e_base, cpw, ne):
    import functools
    mesh = plsc.VectorSubcoreMesh(core_axis_name="c", subcore_axis_name="s")
    bkv = pltpu.VMEM((CHUNK, 2 * D), jnp.float32)
    bq = pltpu.VMEM((CHUNK, D), jnp.float32)
    fn = pl.kernel(
        functools.partial(_gather_body, cpw, e_base),
        out_type=(jax.ShapeDtypeStruct((ne, 2 * D), jnp.float32),
                  jax.ShapeDtypeStruct((ne, D), jnp.float32)),
        mesh=mesh,
        scratch_types=[
            pltpu.VMEM((cpw * CHUNK,), jnp.int32),
            pltpu.VMEM((cpw * CHUNK,), jnp.int32),
            bkv, bq, bkv, bq,
            pltpu.SemaphoreType.DMA,
            pltpu.SemaphoreType.DMA,
            pltpu.SemaphoreType.DMA,
            pltpu.SemaphoreType.DMA,
        ],
    )
    return fn(kvtab, qtab, src_p, dst_p)


def _edge_kernel(kvs_ref, qd_ref, ea_ref, sbf_ref,
                 wekt_ref, wev_ref, ws0_ref, ws1_ref, msg_ref, eal_ref):
    ks = kvs_ref[:, :D]
    vs = kvs_ref[:, D:]
    qd = qd_ref[...]
    ea = ea_ref[...]
    qw = jnp.dot(qd, wekt_ref[...], preferred_element_type=jnp.float32)
    alpha = (jnp.sum(ks * qd, axis=1, keepdims=True)
             + jnp.sum(qw * ea, axis=1, keepdims=True)) * _SQRT_C_INV
    ealpha = jnp.exp(alpha)
    ev = jnp.dot(ea, wev_ref[...], preferred_element_type=jnp.float32)
    sb = jnp.dot(jnp.dot(sbf_ref[...], ws0_ref[...],
                         preferred_element_type=jnp.float32),
                 ws1_ref[...], preferred_element_type=jnp.float32)
    msg_ref[...] = (vs + ev) * sb * ealpha
    eal_ref[...] = ealpha


def _edge_pass(kvsrc, qdst, ea_p, sbf_p, W_ekT, W_ev, W_sbf0, W_sbf1,
               e_base):
    B = 512
    ne = kvsrc.shape[0]
    g = ne // B
    ob = e_base // B
    row = lambda i: (i, 0)
    rowo = lambda i: (i + ob, 0)
    full = lambda i: (0, 0)
    return pl.pallas_call(
        _edge_kernel,
        grid=(g,),
        in_specs=[
            pl.BlockSpec((B, 2 * D), row),
            pl.BlockSpec((B, D), row),
            pl.BlockSpec((B, ED), rowo),
            pl.BlockSpec((B, ED), rowo),
            pl.BlockSpec((D, ED), full),
            pl.BlockSpec((ED, D), full),
            pl.BlockSpec((ED, D), full),
            pl.BlockSpec((D, D), full),
        ],
        out_specs=[
            pl.BlockSpec((B, D), row),
            pl.BlockSpec((B, 1), row),
        ],
        out_shape=[
            jax.ShapeDtypeStruct((ne, D), jnp.float32),
            jax.ShapeDtypeStruct((ne, 1), jnp.float32),
        ],
    )(kvsrc, qdst, ea_p, sbf_p, W_ekT, W_ev, W_sbf0, W_sbf1)


def _scatter_body(cpw, e_base, msg_h, eal_h, dst_h, acc_o, den_o,
                  idx_d0, mbuf0, ebuf0, idx_d1, mbuf1, ebuf1,
                  tbuf, dacc, denom, acc, den_sh,
                  ls0, ls1):
    cid = lax.axis_index("c")
    sid = lax.axis_index("s")
    wid = sid * NC + cid

    if True:
        def zrow(r, _):
            for k in range(D // 16):
                mbuf0[r, pl.ds(k * 16, 16)] = jnp.zeros((16,), jnp.float32)
            return 0

        def initz(s, _):
            r0 = sid * RPS + s * SCH
            pltpu.sync_copy(mbuf0, acc.at[pl.ds(r0, SCH)])
            return 0

        def initd(i, _):
            denom[pl.ds(i * 16, 16)] = jnp.zeros((16,), jnp.float32)
            return 0

        lax.fori_loop(0, SCH, zrow, 0)
        lax.fori_loop(0, RPS // SCH, initz, 0)
        lax.fori_loop(0, NPAD // 16, initd, 0)
        plsc.subcore_barrier()

        slots = ((idx_d0, mbuf0, ebuf0), (idx_d1, mbuf1, ebuf1))
        lsems = (ls0, ls1)

        def fire_loads(j, s):
            lbase = (wid * cpw + j) * SCH
            idx_d, mbuf, ebuf = slots[s]
            pltpu.async_copy(dst_h.at[pl.ds(e_base + lbase, SCH)],
                             idx_d, lsems[s])
            pltpu.async_copy(msg_h.at[pl.ds(lbase, SCH)], mbuf, lsems[s])
            pltpu.async_copy(eal_h.at[pl.ds(lbase, SCH)], ebuf, lsems[s])

        def drain_loads(s):
            idx_d, mbuf, ebuf = slots[s]
            pltpu.make_async_copy(dst_h.at[pl.ds(0, SCH)], idx_d,
                                  lsems[s]).wait()
            pltpu.make_async_copy(msg_h.at[pl.ds(0, SCH)], mbuf,
                                  lsems[s]).wait()
            pltpu.make_async_copy(eal_h.at[pl.ds(0, SCH)], ebuf,
                                  lsems[s]).wait()

        def consume(s):
            idx_d, mbuf, ebuf = slots[s]
            drain_loads(s)
            pltpu.sync_copy(mbuf, acc.at[idx_d], add=True)
            for k in range(SCH // 16):
                i16 = idx_d[pl.ds(k * 16, 16)]
                e16 = ebuf[pl.ds(k * 16, 16)]
                plsc.addupdate_scatter(denom, [i16], e16)

        fire_loads(0, 0)

        def chunk(j, _):
            s = lax.rem(j, 2)
            ns = 1 - s

            @pl.when(j + 1 < cpw)
            def _():
                @pl.when(ns == 0)
                def _():
                    fire_loads(j + 1, 0)

                @pl.when(ns == 1)
                def _():
                    fire_loads(j + 1, 1)

            @pl.when(s == 0)
            def _():
                consume(0)

            @pl.when(s == 1)
            def _():
                consume(1)

            return 0

        lax.fori_loop(0, cpw, chunk, 0)

        pltpu.sync_copy(denom, den_sh.at[sid])
        plsc.subcore_barrier()

        def zslice(i, _):
            dacc[pl.ds(i * 16, 16)] = jnp.zeros((16,), jnp.float32)
            return 0

        lax.fori_loop(0, RPS // 16, zslice, 0)

        def redt(t, _):
            pltpu.sync_copy(den_sh.at[t, pl.ds(sid * RPS, RPS)], tbuf)

            def addv(i, _):
                dacc[pl.ds(i * 16, 16)] = (dacc[pl.ds(i * 16, 16)]
                                           + tbuf[pl.ds(i * 16, 16)])
                return 0

            lax.fori_loop(0, RPS // 16, addv, 0)
            return 0

        lax.fori_loop(0, NS, redt, 0)

        r0 = sid * RPS
        pltpu.sync_copy(acc.at[pl.ds(r0, RPS)],
                        acc_o.at[cid, pl.ds(r0, RPS)])
        pltpu.sync_copy(dacc, den_o.at[cid, pl.ds(r0, RPS)])


def _scatter_pass(msg, eal, dst_p, e_base, cpw):
    import functools
    mesh = plsc.VectorSubcoreMesh(core_axis_name="c", subcore_axis_name="s")
    fn = pl.kernel(
        functools.partial(_scatter_body, cpw, e_base),
        out_type=(
            jax.ShapeDtypeStruct((NC, NPAD, D), jnp.float32),
            jax.ShapeDtypeStruct((NC, NPAD), jnp.float32),
        ),
        mesh=mesh,
        scratch_types=[
            pltpu.VMEM((SCH,), jnp.int32),
            pltpu.VMEM((SCH, D), jnp.float32),
            pltpu.VMEM((SCH,), jnp.float32),
            pltpu.VMEM((SCH,), jnp.int32),
            pltpu.VMEM((SCH, D), jnp.float32),
            pltpu.VMEM((SCH,), jnp.float32),
            pltpu.VMEM((RPS,), jnp.float32),
            pltpu.VMEM((RPS,), jnp.float32),
            pltpu.VMEM((NPAD,), jnp.float32),
            pltpu.VMEM_SHARED((NPAD, D), jnp.float32),
            pltpu.VMEM_SHARED((NS, NPAD), jnp.float32),
            pltpu.SemaphoreType.DMA,
            pltpu.SemaphoreType.DMA,
        ],
        compiler_params=pltpu.CompilerParams(needs_layout_passes=False),
    )
    return fn(msg, eal, dst_p)


def _final_kernel(a0_ref, a1_ref, a2_ref, a3_ref,
                  d0_ref, d1_ref, d2_ref, d3_ref, skip_ref, out_ref):
    a = (a0_ref[0] + a1_ref[0]) + (a2_ref[0] + a3_ref[0])
    den = (d0_ref[0] + d1_ref[0]) + (d2_ref[0] + d3_ref[0]) + 1e-16
    out_ref[...] = a / den + skip_ref[...]


def _final_pass(accsA, accsB, dens3A, dens3B, skip):
    B = 512
    g = pl.cdiv(N, B)
    row = lambda i: (i, 0)
    c0 = lambda i: (0, i, 0)
    c1 = lambda i: (1, i, 0)
    return pl.pallas_call(
        _final_kernel,
        grid=(g,),
        in_specs=[
            pl.BlockSpec((1, B, D), c0),
            pl.BlockSpec((1, B, D), c1),
            pl.BlockSpec((1, B, D), c0),
            pl.BlockSpec((1, B, D), c1),
            pl.BlockSpec((1, B, 1), c0),
            pl.BlockSpec((1, B, 1), c1),
            pl.BlockSpec((1, B, 1), c0),
            pl.BlockSpec((1, B, 1), c1),
            pl.BlockSpec((B, D), row),
        ],
        out_specs=pl.BlockSpec((B, D), row),
        out_shape=jax.ShapeDtypeStruct((N, D), jnp.float32),
    )(accsA, accsA, accsB, accsB, dens3A, dens3A, dens3B, dens3B, skip)


def kernel(x, edge_index, edge_attr, rbf, sbf, edge_index_0,
           W_rbf0, W_rbf1, W_sbf0, W_sbf1, W_ek, W_ev,
           W_k, b_k, W_q, b_q, W_v, b_v, W_skip, b_skip):
    pad_e = E2 - E
    src_p = jnp.concatenate(
        [edge_index[0], jnp.zeros((pad_e,), jnp.int32)])
    dst_p = jnp.concatenate(
        [edge_index[1], jnp.full((pad_e,), N, jnp.int32)])
    ea_p = edge_attr
    sbf_p = sbf.reshape(E, ED)
    x_p = jnp.pad(x, ((0, NPAD - N), (0, 0)))
    rbf_p = jnp.pad(rbf, ((0, NPAD - N), (0, 0)))

    kvtab, qtab, skip = _node_pass(
        x_p, rbf_p, W_rbf0, W_rbf1, W_k, b_k, W_q, b_q, W_v, b_v,
        W_skip, b_skip)

    CPW_A, CPW_B = 40, 39
    EA_N = NW * CPW_A * CHUNK
    EB_N = NW * CPW_B * CHUNK
    W_ekT = W_ek.T

    gA = _gather_pass(kvtab, qtab, src_p, dst_p, 0, CPW_A, EA_N)
    gB = _gather_pass(kvtab, qtab, src_p, dst_p, EA_N, CPW_B, EB_N)

    msgA, ealA = _edge_pass(*gA, ea_p, sbf_p, W_ekT, W_ev, W_sbf0, W_sbf1, 0)
    msgB, ealB = _edge_pass(*gB, ea_p, sbf_p, W_ekT, W_ev, W_sbf0, W_sbf1,
                            EA_N)

    accsA, densA = _scatter_pass(msgA, ealA.reshape(EA_N), dst_p, 0,
                                 CPW_A * CHUNK // SCH)
    accsB, densB = _scatter_pass(msgB, ealB.reshape(EB_N), dst_p, EA_N,
                                 CPW_B * CHUNK // SCH)

    return _final_pass(accsA, accsB,
                       densA.reshape(NC, NPAD, 1), densB.reshape(NC, NPAD, 1),
                       skip)
